# Initial kernel scaffold; baseline (speedup 1.0000x reference)
import jax
import jax.numpy as jnp
from jax.experimental import pallas as pl


def kernel(h, x, edge_index, edge_attr, params):
    raise NotImplementedError("write your pallas kernel here")


# speedup vs baseline: 2.0046x; 2.0046x over previous
import functools

import jax
import jax.numpy as jnp
from jax import lax
from jax.experimental import pallas as pl
from jax.experimental.pallas import tpu as pltpu
import jax.experimental.pallas.tpu_sc as plsc

N = 10000
E = 320000
H = 128

NC = 2
NS = 16
NW = NC * NS
EPW = E // NW
CH = 80
NCHUNK = EPW // CH
RPT = N // NS

def _mesh():
    return plsc.VectorSubcoreMesh(
        core_axis_name="c", subcore_axis_name="s",
        num_cores=NC, num_subcores=NS)

_f32 = jnp.float32


def _sc_gather0_body(pr, pc, xx, xy, xz, row, col,
                     gr, gc, oxr, oyr, ozr, oxc, oyc, ozc,
                     idx_r, idx_c, bufr, bufc,
                     bxr, byr, bzr, bxc, byc, bzc, sem):
    wid = lax.axis_index("s") * NC + lax.axis_index("c")
    base = wid * EPW

    def chunk(i, carry):
        off = base + i * CH
        pltpu.sync_copy(row.at[pl.ds(off, CH)], idx_r)
        pltpu.sync_copy(col.at[pl.ds(off, CH)], idx_c)
        cps = [
            pltpu.async_copy(pr.at[idx_r], bufr, sem),
            pltpu.async_copy(pc.at[idx_c], bufc, sem),
            pltpu.async_copy(xx.at[idx_r], bxr, sem),
            pltpu.async_copy(xy.at[idx_r], byr, sem),
            pltpu.async_copy(xz.at[idx_r], bzr, sem),
            pltpu.async_copy(xx.at[idx_c], bxc, sem),
            pltpu.async_copy(xy.at[idx_c], byc, sem),
            pltpu.async_copy(xz.at[idx_c], bzc, sem),
        ]
        for cp in cps:
            cp.wait()
        pltpu.sync_copy(bufr, gr.at[pl.ds(off, CH)])
        pltpu.sync_copy(bufc, gc.at[pl.ds(off, CH)])
        pltpu.sync_copy(bxr, oxr.at[pl.ds(off, CH)])
        pltpu.sync_copy(byr, oyr.at[pl.ds(off, CH)])
        pltpu.sync_copy(bzr, ozr.at[pl.ds(off, CH)])
        pltpu.sync_copy(bxc, oxc.at[pl.ds(off, CH)])
        pltpu.sync_copy(byc, oyc.at[pl.ds(off, CH)])
        pltpu.sync_copy(bzc, ozc.at[pl.ds(off, CH)])
        return carry

    lax.fori_loop(0, NCHUNK, chunk, 0)


@functools.cache
def _gather0():
    vE = jax.ShapeDtypeStruct((E,), _f32)
    bC = pltpu.VMEM((CH,), _f32)
    return pl.kernel(
        _sc_gather0_body,
        out_type=[
            jax.ShapeDtypeStruct((E, H), _f32),
            jax.ShapeDtypeStruct((E, H), _f32),
            vE, vE, vE, vE, vE, vE,
        ],
        mesh=_mesh(),
        scratch_types=[
            pltpu.VMEM((CH,), jnp.int32),
            pltpu.VMEM((CH,), jnp.int32),
            pltpu.VMEM((CH, H), _f32),
            pltpu.VMEM((CH, H), _f32),
            bC, bC, bC, bC, bC, bC,
            pltpu.SemaphoreType.DMA,
        ],
    )


def _sc_gather2_body(pr, pc, row, col, gr, gc, idx_r, idx_c, bufr, bufc, sem):
    wid = lax.axis_index("s") * NC + lax.axis_index("c")
    base = wid * EPW

    def chunk(i, carry):
        off = base + i * CH
        pltpu.sync_copy(row.at[pl.ds(off, CH)], idx_r)
        pltpu.sync_copy(col.at[pl.ds(off, CH)], idx_c)
        cp1 = pltpu.async_copy(pr.at[idx_r], bufr, sem)
        cp2 = pltpu.async_copy(pc.at[idx_c], bufc, sem)
        cp1.wait(); cp2.wait()
        pltpu.sync_copy(bufr, gr.at[pl.ds(off, CH)])
        pltpu.sync_copy(bufc, gc.at[pl.ds(off, CH)])
        return carry

    lax.fori_loop(0, NCHUNK, chunk, 0)


@functools.cache
def _gather2():
    return pl.kernel(
        _sc_gather2_body,
        out_type=[
            jax.ShapeDtypeStruct((E, H), _f32),
            jax.ShapeDtypeStruct((E, H), _f32),
        ],
        mesh=_mesh(),
        scratch_types=[
            pltpu.VMEM((CH,), jnp.int32),
            pltpu.VMEM((CH,), jnp.int32),
            pltpu.VMEM((CH, H), _f32),
            pltpu.VMEM((CH, H), _f32),
            pltpu.SemaphoreType.DMA,
        ],
    )


@functools.cache
def _make_scatter(W):
    def body(vals, row, zeros, agg, idx, buf, acc, sem):
        cid = lax.axis_index("c")
        sid = lax.axis_index("s")
        base = (cid * NS + sid) * EPW

        @pl.when(sid == 0)
        def _():
            pltpu.sync_copy(zeros, acc)

        plsc.subcore_barrier()

        def chunk(i, carry):
            off = base + i * CH
            pltpu.sync_copy(row.at[pl.ds(off, CH)], idx)
            pltpu.sync_copy(vals.at[pl.ds(off, CH)], buf)
            pltpu.sync_copy(buf, acc.at[idx], add=True)
            return carry

        lax.fori_loop(0, NCHUNK, chunk, 0)
        plsc.subcore_barrier()

        @pl.when(sid < NS - 1)
        def _():
            pltpu.sync_copy(acc.at[pl.ds(sid * 624, 624)],
                            agg.at[cid, pl.ds(sid * 624, 624)])

        @pl.when(sid == NS - 1)
        def _():
            pltpu.sync_copy(acc.at[pl.ds((NS - 1) * 624, N - (NS - 1) * 624)],
                            agg.at[cid, pl.ds((NS - 1) * 624, N - (NS - 1) * 624)])

    return pl.kernel(
        body,
        out_type=jax.ShapeDtypeStruct((NC, N, W), _f32),
        mesh=_mesh(),
        scratch_types=[
            pltpu.VMEM((CH,), jnp.int32),
            pltpu.VMEM((CH, W), _f32),
            pltpu.VMEM_SHARED((N, W), _f32),
            pltpu.SemaphoreType.DMA,
        ],
    )


def _scatter_h(vals, row, zeros):
    return _make_scatter(H)(vals, row, zeros)


def _sc_scatter3_body(tx, ty, tz, row, zeros1, aggx, aggy, aggz,
                      idx, bx, by, bz, accx, accy, accz, bo, sem):
    cid = lax.axis_index("c")
    sid = lax.axis_index("s")
    base = (cid * NS + sid) * EPW

    @pl.when(sid == 0)
    def _():
        pltpu.sync_copy(zeros1, accx)
        pltpu.sync_copy(zeros1, accy)
        pltpu.sync_copy(zeros1, accz)

    plsc.subcore_barrier()

    def chunk(i, carry):
        off = base + i * CH
        pltpu.sync_copy(row.at[pl.ds(off, CH)], idx)
        pltpu.sync_copy(tx.at[pl.ds(off, CH)], bx)
        pltpu.sync_copy(ty.at[pl.ds(off, CH)], by)
        pltpu.sync_copy(tz.at[pl.ds(off, CH)], bz)
        pltpu.sync_copy(bx, accx.at[idx], add=True)
        pltpu.sync_copy(by, accy.at[idx], add=True)
        pltpu.sync_copy(bz, accz.at[idx], add=True)
        return carry

    lax.fori_loop(0, NCHUNK, chunk, 0)
    plsc.subcore_barrier()

    sz0 = 624
    szL = N - (NS - 1) * sz0

    def wout(acc, agg, start, sz):
        pltpu.sync_copy(acc.at[pl.ds(start, sz)], bo.at[pl.ds(0, sz)])
        pltpu.sync_copy(bo.at[pl.ds(0, sz)], agg.at[pl.ds(cid * N + start, sz)])

    @pl.when(sid < NS - 1)
    def _():
        wout(accx, aggx, sid * sz0, sz0)
        wout(accy, aggy, sid * sz0, sz0)
        wout(accz, aggz, sid * sz0, sz0)

    @pl.when(sid == NS - 1)
    def _():
        wout(accx, aggx, (NS - 1) * sz0, szL)
        wout(accy, aggy, (NS - 1) * sz0, szL)
        wout(accz, aggz, (NS - 1) * sz0, szL)


@functools.cache
def _scatter3():
    vN = jax.ShapeDtypeStruct((NC * N,), _f32)
    bC = pltpu.VMEM((CH,), _f32)
    aN = pltpu.VMEM_SHARED((N,), _f32)
    return pl.kernel(
        _sc_scatter3_body,
        out_type=[vN, vN, vN],
        mesh=_mesh(),
        scratch_types=[
            pltpu.VMEM((CH,), jnp.int32),
            bC, bC, bC,
            aN, aN, aN,
            pltpu.VMEM((640,), _f32),
            pltpu.SemaphoreType.DMA,
        ],
    )


BN = 2000
BE = 3200


def _rows(bs, w):
    return pl.BlockSpec((bs, w), lambda i: (i, 0))


def _full(shape):
    return pl.BlockSpec(shape, lambda i: tuple(0 for _ in shape))


def _silu(v):
    return v * jax.nn.sigmoid(v)


def _dot(a, b):
    return jnp.dot(a, b, preferred_element_type=_f32)


def _tc_proj_body(h_ref, wr_ref, wc_ref, b_ref, pr_ref, pc_ref):
    hb = h_ref[...]
    pr_ref[...] = _dot(hb, wr_ref[...]) + b_ref[...]
    pc_ref[...] = _dot(hb, wc_ref[...])


def _proj(h, wr, wc, b):
    return pl.pallas_call(
        _tc_proj_body,
        grid=(N // BN,),
        in_specs=[_rows(BN, H), _full((H, H)), _full((H, H)), _full((1, H))],
        out_specs=[_rows(BN, H), _rows(BN, H)],
        out_shape=[jax.ShapeDtypeStruct((N, H), _f32)] * 2,
    )(h, wr, wc, b)


def _tc_geom_body(xr_ref, yr_ref, zr_ref, xc_ref, yc_ref, zc_ref, ea_ref,
                  e8_ref):
    cdx = xr_ref[...] - xc_ref[...]
    cdy = yr_ref[...] - yc_ref[...]
    cdz = zr_ref[...] - zc_ref[...]
    radial = cdx * cdx + cdy * cdy + cdz * cdz
    inv = 1.0 / (jnp.sqrt(radial + 1e-8) + 1.0)
    z = jnp.zeros_like(radial)
    e8_ref[...] = jnp.concatenate(
        [radial, ea_ref[...], cdx * inv, cdy * inv, cdz * inv, z, z, z],
        axis=1)


def _geom(xr, yr, zr, xc, yc, zc, ea):
    return pl.pallas_call(
        _tc_geom_body,
        grid=(E // BE,),
        in_specs=[_rows(BE, 1)] * 7,
        out_specs=_rows(BE, 8),
        out_shape=jax.ShapeDtypeStruct((E, 8), _f32),
    )(xr, yr, zr, xc, yc, zc, ea)


def _tc_edge_gcl_body(gr_ref, gc_ref, e8_ref, w2_ref, b2_ref, wa_ref, ba_ref,
                      wre_ref, out_ref):
    e8 = e8_ref[...]
    wre = wre_ref[...]
    v = gr_ref[...] + gc_ref[...] + e8[:, 0:1] * wre[0:1, :] + e8[:, 1:2] * wre[1:2, :]
    m1 = _silu(v)
    mm = _dot(m1, w2_ref[...]) + b2_ref[...]
    m = _silu(mm)
    att = jax.nn.sigmoid(_dot(m, wa_ref[...]) + ba_ref[...])
    out_ref[...] = m * att


def _edge_gcl(gr, gc, e8, w2, b2, wa, ba, wre):
    return pl.pallas_call(
        _tc_edge_gcl_body,
        grid=(E // BE,),
        in_specs=[_rows(BE, H), _rows(BE, H), _rows(BE, 8),
                  _full((H, H)), _full((1, H)), _full((H, 1)), _full((1, 1)),
                  _full((2, H))],
        out_specs=_rows(BE, H),
        out_shape=jax.ShapeDtypeStruct((E, H), _f32),
    )(gr, gc, e8, w2, b2, wa, ba, wre)


def _tc_edge_equiv_body(gr_ref, gc_ref, e8_ref, w2_ref, b2_ref, w3_ref,
                        wre_ref, tx_ref, ty_ref, tz_ref):
    e8 = e8_ref[...]
    wre = wre_ref[...]
    v = gr_ref[...] + gc_ref[...] + e8[:, 0:1] * wre[0:1, :] + e8[:, 1:2] * wre[1:2, :]
    t1 = _silu(v)
    t2 = _silu(_dot(t1, w2_ref[...]) + b2_ref[...])
    t = _dot(t2, w3_ref[...])
    tx_ref[...] = e8[:, 2:3] * t
    ty_ref[...] = e8[:, 3:4] * t
    tz_ref[...] = e8[:, 4:5] * t


def _edge_equiv(gr, gc, e8, w2, b2, w3, wre):
    return pl.pallas_call(
        _tc_edge_equiv_body,
        grid=(E // BE,),
        in_specs=[_rows(BE, H), _rows(BE, H), _rows(BE, 8),
                  _full((H, H)), _full((1, H)), _full((H, 1)), _full((2, H))],
        out_specs=[_rows(BE, 1)] * 3,
        out_shape=[jax.ShapeDtypeStruct((E, 1), _f32)] * 3,
    )(gr, gc, e8, w2, b2, w3, wre)


def _tc_node_body(h_ref, a0_ref, a1_ref, wna_ref, wnb_ref, bn1_ref, wn2_ref,
                  bn2_ref, wrn_ref, wcn_ref, brn_ref,
                  hout_ref, pr_ref, pc_ref):
    hb = h_ref[...]
    agg = a0_ref[...] + a1_ref[...]
    pre = _dot(hb, wna_ref[...]) + _dot(agg, wnb_ref[...]) + bn1_ref[...]
    n1 = _silu(pre)
    ho = hb + _dot(n1, wn2_ref[...]) + bn2_ref[...]
    hout_ref[...] = ho
    pr_ref[...] = _dot(ho, wrn_ref[...]) + brn_ref[...]
    pc_ref[...] = _dot(ho, wcn_ref[...])


def _node(h, a0, a1, wna, wnb, bn1, wn2, bn2, wrn, wcn, brn):
    return pl.pallas_call(
        _tc_node_body,
        grid=(N // BN,),
        in_specs=[_rows(BN, H), _rows(BN, H), _rows(BN, H),
                  _full((H, H)), _full((H, H)), _full((1, H)),
                  _full((H, H)), _full((1, H)),
                  _full((H, H)), _full((H, H)), _full((1, H))],
        out_specs=[_rows(BN, H)] * 3,
        out_shape=[jax.ShapeDtypeStruct((N, H), _f32)] * 3,
    )(h, a0, a1, wna, wnb, bn1, wn2, bn2, wrn, wcn, brn)


def _tc_final_body(x_ref, ax0, ax1, ay0, ay1, az0, az1, xout_ref):
    agg = jnp.concatenate([ax0[...] + ax1[...], ay0[...] + ay1[...],
                           az0[...] + az1[...]], axis=1)
    xout_ref[...] = x_ref[...] + agg * 0.01


def _final(x, ax0, ax1, ay0, ay1, az0, az1):
    return pl.pallas_call(
        _tc_final_body,
        grid=(N // BN,),
        in_specs=[_rows(BN, 3)] + [_rows(BN, 1)] * 6,
        out_specs=_rows(BN, 3),
        out_shape=jax.ShapeDtypeStruct((N, 3), _f32),
    )(x, ax0, ax1, ay0, ay1, az0, az1)


def _split_edge_w(lin):
    w = lin["w"]
    wre = jnp.stack([w[2 * H], w[2 * H + 1]], axis=0)
    return w[:H], w[H:2 * H], wre, lin["b"].reshape(1, H)


def kernel(h, x, edge_index, edge_attr, params):
    row = edge_index[0]
    col = edge_index[1]
    xx = x[:, 0]
    xy = x[:, 1]
    xz = x[:, 2]

    g0, g1, pe = params["gcl0"], params["gcl1"], params["equiv"]
    w1r0, w1c0, wre0, b10 = _split_edge_w(g0["edge1"])
    w1r1, w1c1, wre1, b11 = _split_edge_w(g1["edge1"])
    c1r, c1c, wree, bc1 = _split_edge_w(pe["c1"])

    zeros_h = jnp.zeros((N, H), _f32)

    def node_w(g):
        wn1 = g["node1"]["w"]
        return (wn1[:H], wn1[H:] * 0.01, g["node1"]["b"].reshape(1, H),
                g["node2"]["w"], g["node2"]["b"].reshape(1, H))

    pr0, pc0 = _proj(h, w1r0, w1c0, b10)
    gr0, gc0, oxr, oyr, ozr, oxc, oyc, ozc = _gather0()(
        pr0, pc0, xx, xy, xz, row, col)
    e8 = _geom(oxr.reshape(E, 1), oyr.reshape(E, 1), ozr.reshape(E, 1),
               oxc.reshape(E, 1), oyc.reshape(E, 1), ozc.reshape(E, 1),
               edge_attr)
    out0 = _edge_gcl(gr0, gc0, e8, g0["edge2"]["w"],
                     g0["edge2"]["b"].reshape(1, H), g0["att"]["w"],
                     g0["att"]["b"].reshape(1, 1), wre0)
    agg0 = _scatter_h(out0, row, zeros_h)
    wna, wnb, bn1, wn2, bn2 = node_w(g0)
    h1, pr1, pc1 = _node(h, agg0[0], agg0[1], wna, wnb, bn1, wn2, bn2,
                         w1r1, w1c1, b11)

    gr1, gc1 = _gather2()(pr1, pc1, row, col)
    out1 = _edge_gcl(gr1, gc1, e8, g1["edge2"]["w"],
                     g1["edge2"]["b"].reshape(1, H), g1["att"]["w"],
                     g1["att"]["b"].reshape(1, 1), wre1)
    agg1 = _scatter_h(out1, row, zeros_h)
    wna, wnb, bn1, wn2, bn2 = node_w(g1)
    h2, qr, qc = _node(h1, agg1[0], agg1[1], wna, wnb, bn1, wn2, bn2,
                       c1r, c1c, bc1)

    gqr, gqc = _gather2()(qr, qc, row, col)
    etx, ety, etz = _edge_equiv(gqr, gqc, e8, pe["c2"]["w"],
                                pe["c2"]["b"].reshape(1, H), pe["c3w"], wree)
    zeros_1 = jnp.zeros((N,), _f32)
    ax, ay, az = _scatter3()(etx.reshape(E), ety.reshape(E), etz.reshape(E),
                             row, zeros_1)
    x_out = _final(x,
                   ax[:N].reshape(N, 1), ax[N:].reshape(N, 1),
                   ay[:N].reshape(N, 1), ay[N:].reshape(N, 1),
                   az[:N].reshape(N, 1), az[N:].reshape(N, 1))

    return (h2, x_out)


# speedup vs baseline: 4.9856x; 2.4871x over previous
import functools

import jax
import jax.numpy as jnp
from jax import lax
from jax.experimental import pallas as pl
from jax.experimental.pallas import tpu as pltpu
import jax.experimental.pallas.tpu_sc as plsc

N = 10000
E = 320000
H = 128
ER = E // H

NC = 2
NS = 16
NW = NC * NS
EPW = E // NW
CH = 40
NCHUNK = EPW // CH
NSLOT = 5
NGRP = NCHUNK // NSLOT


def _mesh():
    return plsc.VectorSubcoreMesh(
        core_axis_name="c", subcore_axis_name="s",
        num_cores=NC, num_subcores=NS)


_f32 = jnp.float32


def _sc_gather0_body(pr, pc, xx, xy, xz, row, col,
                     gr, gc, oxr, oyr, ozr, oxc, oyc, ozc,
                     idx_r, idx_c, bufs, sems):
    wid = lax.axis_index("s") * NC + lax.axis_index("c")
    base = wid * EPW
    pltpu.sync_copy(row.at[pl.ds(base, EPW)], idx_r)
    pltpu.sync_copy(col.at[pl.ds(base, EPW)], idx_c)

    bufr, bufc, bxr, byr, bzr, bxc, byc, bzc = bufs
    gsem, ssem = sems

    def gops(i, s):
        ir = idx_r.at[pl.ds(i * CH, CH)]
        ic = idx_c.at[pl.ds(i * CH, CH)]
        return [
            (pr.at[ir], bufr.at[s]), (pc.at[ic], bufc.at[s]),
            (xx.at[ir], bxr.at[s]), (xy.at[ir], byr.at[s]),
            (xz.at[ir], bzr.at[s]),
            (xx.at[ic], bxc.at[s]), (xy.at[ic], byc.at[s]),
            (xz.at[ic], bzc.at[s]),
        ]

    def sops(s, off):
        d = pl.ds(off, CH)
        return [
            (bufr.at[s], gr.at[d]), (bufc.at[s], gc.at[d]),
            (bxr.at[s], oxr.at[d]), (byr.at[s], oyr.at[d]),
            (bzr.at[s], ozr.at[d]),
            (bxc.at[s], oxc.at[d]), (byc.at[s], oyc.at[d]),
            (bzc.at[s], ozc.at[d]),
        ]

    def fire(i, s):
        for a, b in gops(i, s):
            pltpu.async_copy(a, b, gsem[s])

    for s in range(NSLOT):
        fire(s, s)

    def group(g, carry):
        for s in range(NSLOT):
            i = g * NSLOT + s
            off = base + i * CH
            for a, b in gops(i, s):
                pltpu.make_async_copy(a, b, gsem[s]).wait()
            for a, b in sops(s, off):
                pltpu.async_copy(a, b, ssem[s])
            for a, b in sops(s, off):
                pltpu.make_async_copy(a, b, ssem[s]).wait()

            @pl.when(g < NGRP - 1)
            def _():
                fire(i + NSLOT, s)
        return carry

    lax.fori_loop(0, NGRP, group, 0)


@functools.cache
def _gather0():
    vE = jax.ShapeDtypeStruct((E,), _f32)
    bC = pltpu.VMEM((NSLOT, CH), _f32)
    return pl.kernel(
        _sc_gather0_body,
        out_type=[
            jax.ShapeDtypeStruct((E, H), _f32),
            jax.ShapeDtypeStruct((E, H), _f32),
            vE, vE, vE, vE, vE, vE,
        ],
        mesh=_mesh(),
        scratch_types=[
            pltpu.VMEM((EPW,), jnp.int32),
            pltpu.VMEM((EPW,), jnp.int32),
            (pltpu.VMEM((NSLOT, CH, H), _f32), pltpu.VMEM((NSLOT, CH, H), _f32),
             bC, bC, bC, bC, bC, bC),
            ([pltpu.SemaphoreType.DMA] * NSLOT, [pltpu.SemaphoreType.DMA] * NSLOT),
        ],
    )


def _sc_gather2_body(pr, pc, row, col, gr, gc, idx_r, idx_c, bufs, sems):
    wid = lax.axis_index("s") * NC + lax.axis_index("c")
    base = wid * EPW
    pltpu.sync_copy(row.at[pl.ds(base, EPW)], idx_r)
    pltpu.sync_copy(col.at[pl.ds(base, EPW)], idx_c)

    bufr, bufc = bufs
    gsem, ssem = sems

    def fire(i, s):
        pltpu.async_copy(pr.at[idx_r.at[pl.ds(i * CH, CH)]], bufr.at[s], gsem[s])
        pltpu.async_copy(pc.at[idx_c.at[pl.ds(i * CH, CH)]], bufc.at[s], gsem[s])

    for s in range(NSLOT):
        fire(s, s)

    def group(g, carry):
        for s in range(NSLOT):
            i = g * NSLOT + s
            off = base + i * CH
            ir = idx_r.at[pl.ds(i * CH, CH)]
            ic = idx_c.at[pl.ds(i * CH, CH)]
            pltpu.make_async_copy(pr.at[ir], bufr.at[s], gsem[s]).wait()
            pltpu.make_async_copy(pc.at[ic], bufc.at[s], gsem[s]).wait()
            pltpu.async_copy(bufr.at[s], gr.at[pl.ds(off, CH)], ssem[s])
            pltpu.async_copy(bufc.at[s], gc.at[pl.ds(off, CH)], ssem[s])
            pltpu.make_async_copy(bufr.at[s], gr.at[pl.ds(off, CH)], ssem[s]).wait()
            pltpu.make_async_copy(bufc.at[s], gc.at[pl.ds(off, CH)], ssem[s]).wait()

            @pl.when(g < NGRP - 1)
            def _():
                fire(i + NSLOT, s)
        return carry

    lax.fori_loop(0, NGRP, group, 0)


@functools.cache
def _gather2():
    return pl.kernel(
        _sc_gather2_body,
        out_type=[
            jax.ShapeDtypeStruct((E, H), _f32),
            jax.ShapeDtypeStruct((E, H), _f32),
        ],
        mesh=_mesh(),
        scratch_types=[
            pltpu.VMEM((EPW,), jnp.int32),
            pltpu.VMEM((EPW,), jnp.int32),
            (pltpu.VMEM((NSLOT, CH, H), _f32), pltpu.VMEM((NSLOT, CH, H), _f32)),
            ([pltpu.SemaphoreType.DMA] * NSLOT, [pltpu.SemaphoreType.DMA] * NSLOT),
        ],
    )


@functools.cache
def _make_scatter(W):
    def body(vals, row, zeros, agg, idx, buf, acc, sems):
        cid = lax.axis_index("c")
        sid = lax.axis_index("s")
        wid = cid * NS + sid
        base = wid * EPW
        gsem, ssem = sems

        @pl.when(sid == 0)
        def _():
            pltpu.sync_copy(zeros, acc)

        plsc.subcore_barrier()

        def fire(i, s):
            off = base + i * CH
            pltpu.async_copy(row.at[pl.ds(off, CH)], idx.at[s], gsem[s])
            pltpu.async_copy(vals.at[pl.ds(off, CH)], buf.at[s], gsem[s])

        for s in range(NSLOT):
            fire(s, s)

        def group(g, carry):
            for s in range(NSLOT):
                i = g * NSLOT + s
                off = base + i * CH
                pltpu.make_async_copy(
                    row.at[pl.ds(off, CH)], idx.at[s], gsem[s]).wait()
                pltpu.make_async_copy(
                    vals.at[pl.ds(off, CH)], buf.at[s], gsem[s]).wait()
                cp = pltpu.async_copy(buf.at[s], acc.at[idx.at[s]], ssem[s],
                                      add=True)
                cp.wait()

                @pl.when(g < NGRP - 1)
                def _():
                    fire(i + NSLOT, s)
            return carry

        lax.fori_loop(0, NGRP, group, 0)
        plsc.subcore_barrier()

        @pl.when(sid < NS - 1)
        def _():
            pltpu.sync_copy(acc.at[pl.ds(sid * 624, 624)],
                            agg.at[cid, pl.ds(sid * 624, 624)])

        @pl.when(sid == NS - 1)
        def _():
            pltpu.sync_copy(acc.at[pl.ds((NS - 1) * 624, N - (NS - 1) * 624)],
                            agg.at[cid, pl.ds((NS - 1) * 624, N - (NS - 1) * 624)])

    return pl.kernel(
        body,
        out_type=jax.ShapeDtypeStruct((NC, N, W), _f32),
        mesh=_mesh(),
        scratch_types=[
            pltpu.VMEM((NSLOT, CH), jnp.int32),
            pltpu.VMEM((NSLOT, CH, W), _f32),
            pltpu.VMEM_SHARED((N, W), _f32),
            ([pltpu.SemaphoreType.DMA] * NSLOT, [pltpu.SemaphoreType.DMA] * NSLOT),
        ],
    )


def _scatter_h(vals, row, zeros):
    return _make_scatter(H)(vals, row, zeros)


def _sc_scatter3_body(tx, ty, tz, row, zeros1, aggx, aggy, aggz,
                      idx, bufs, accx, accy, accz, bo, sems):
    cid = lax.axis_index("c")
    sid = lax.axis_index("s")
    wid = cid * NS + sid
    base = wid * EPW
    bx, by, bz = bufs
    gsem, ssem = sems

    @pl.when(sid == 0)
    def _():
        pltpu.sync_copy(zeros1, accx)
        pltpu.sync_copy(zeros1, accy)
        pltpu.sync_copy(zeros1, accz)

    plsc.subcore_barrier()

    def fire(i, s):
        off = base + i * CH
        pltpu.async_copy(row.at[pl.ds(off, CH)], idx.at[s], gsem[s])
        pltpu.async_copy(tx.at[pl.ds(off, CH)], bx.at[s], gsem[s])
        pltpu.async_copy(ty.at[pl.ds(off, CH)], by.at[s], gsem[s])
        pltpu.async_copy(tz.at[pl.ds(off, CH)], bz.at[s], gsem[s])

    for s in range(NSLOT):
        fire(s, s)

    def group(g, carry):
        for s in range(NSLOT):
            i = g * NSLOT + s
            off = base + i * CH
            pltpu.make_async_copy(row.at[pl.ds(off, CH)], idx.at[s], gsem[s]).wait()
            pltpu.make_async_copy(tx.at[pl.ds(off, CH)], bx.at[s], gsem[s]).wait()
            pltpu.make_async_copy(ty.at[pl.ds(off, CH)], by.at[s], gsem[s]).wait()
            pltpu.make_async_copy(tz.at[pl.ds(off, CH)], bz.at[s], gsem[s]).wait()
            pltpu.async_copy(bx.at[s], accx.at[idx.at[s]], ssem[s], add=True)
            pltpu.async_copy(by.at[s], accy.at[idx.at[s]], ssem[s], add=True)
            pltpu.async_copy(bz.at[s], accz.at[idx.at[s]], ssem[s], add=True)
            pltpu.make_async_copy(bx.at[s], accx.at[idx.at[s]], ssem[s]).wait()
            pltpu.make_async_copy(by.at[s], accy.at[idx.at[s]], ssem[s]).wait()
            pltpu.make_async_copy(bz.at[s], accz.at[idx.at[s]], ssem[s]).wait()

            @pl.when(g < NGRP - 1)
            def _():
                fire(i + NSLOT, s)
        return carry

    lax.fori_loop(0, NGRP, group, 0)
    plsc.subcore_barrier()

    sz0 = 624
    szL = N - (NS - 1) * sz0

    def wout(acc, agg, start, sz):
        pltpu.sync_copy(acc.at[pl.ds(start, sz)], bo.at[pl.ds(0, sz)])
        pltpu.sync_copy(bo.at[pl.ds(0, sz)], agg.at[pl.ds(cid * N + start, sz)])

    @pl.when(sid < NS - 1)
    def _():
        wout(accx, aggx, sid * sz0, sz0)
        wout(accy, aggy, sid * sz0, sz0)
        wout(accz, aggz, sid * sz0, sz0)

    @pl.when(sid == NS - 1)
    def _():
        wout(accx, aggx, (NS - 1) * sz0, szL)
        wout(accy, aggy, (NS - 1) * sz0, szL)
        wout(accz, aggz, (NS - 1) * sz0, szL)


@functools.cache
def _scatter3():
    vN = jax.ShapeDtypeStruct((NC * N,), _f32)
    bC = pltpu.VMEM((NSLOT, CH), _f32)
    aN = pltpu.VMEM_SHARED((N,), _f32)
    return pl.kernel(
        _sc_scatter3_body,
        out_type=[vN, vN, vN],
        mesh=_mesh(),
        scratch_types=[
            pltpu.VMEM((NSLOT, CH), jnp.int32),
            (bC, bC, bC),
            aN, aN, aN,
            pltpu.VMEM((640,), _f32),
            ([pltpu.SemaphoreType.DMA] * NSLOT, [pltpu.SemaphoreType.DMA] * NSLOT),
        ],
    )


BN = 2000
BE = 3200
BEL = BE // H


def _rows(bs, w):
    return pl.BlockSpec((bs, w), lambda i: (i, 0))


def _full(shape):
    return pl.BlockSpec(shape, lambda i: tuple(0 for _ in shape))


def _silu(v):
    return v * jax.nn.sigmoid(v)


def _dot(a, b):
    return jnp.dot(a, b, preferred_element_type=_f32)


def _tc_proj_body(h_ref, wr_ref, wc_ref, b_ref, pr_ref, pc_ref):
    hb = h_ref[...]
    pr_ref[...] = _dot(hb, wr_ref[...]) + b_ref[...]
    pc_ref[...] = _dot(hb, wc_ref[...])


def _proj(h, wr, wc, b):
    return pl.pallas_call(
        _tc_proj_body,
        grid=(N // BN,),
        in_specs=[_rows(BN, H), _full((H, H)), _full((H, H)), _full((1, H))],
        out_specs=[_rows(BN, H), _rows(BN, H)],
        out_shape=[jax.ShapeDtypeStruct((N, H), _f32)] * 2,
    )(h, wr, wc, b)


def _tc_geom_body(xr_ref, yr_ref, zr_ref, xc_ref, yc_ref, zc_ref,
                  rad_ref, cnx_ref, cny_ref, cnz_ref):
    cdx = xr_ref[...] - xc_ref[...]
    cdy = yr_ref[...] - yc_ref[...]
    cdz = zr_ref[...] - zc_ref[...]
    radial = cdx * cdx + cdy * cdy + cdz * cdz
    inv = 1.0 / (jnp.sqrt(radial + 1e-8) + 1.0)
    rad_ref[...] = radial
    cnx_ref[...] = cdx * inv
    cny_ref[...] = cdy * inv
    cnz_ref[...] = cdz * inv


def _geom(xr, yr, zr, xc, yc, zc):
    s = _full((ER, H))
    o = jax.ShapeDtypeStruct((ER, H), _f32)
    return pl.pallas_call(
        _tc_geom_body,
        grid=(1,),
        in_specs=[s] * 6,
        out_specs=[s] * 4,
        out_shape=[o] * 4,
    )(xr, yr, zr, xc, yc, zc)


def _eterm(e2blk, wre):
    return lax.dot_general(e2blk, wre, (((0,), (0,)), ((), ())),
                           preferred_element_type=_f32)


def _tc_edge_gcl_body(gr_ref, gc_ref, e2_ref, w2_ref, b2_ref, wa_ref, ba_ref,
                      wre_ref, out_ref):
    v = gr_ref[...] + gc_ref[...] + _eterm(e2_ref[...], wre_ref[...])
    m1 = _silu(v)
    mm = _dot(m1, w2_ref[...]) + b2_ref[...]
    m = _silu(mm)
    att = jax.nn.sigmoid(_dot(m, wa_ref[...]) + ba_ref[...])
    out_ref[...] = m * att


def _edge_gcl(gr, gc, e2, w2, b2, wa, ba, wre):
    return pl.pallas_call(
        _tc_edge_gcl_body,
        grid=(E // BE,),
        in_specs=[_rows(BE, H), _rows(BE, H),
                  pl.BlockSpec((2, BE), lambda i: (0, i)),
                  _full((H, H)), _full((1, H)), _full((H, 1)), _full((1, 1)),
                  _full((2, H))],
        out_specs=_rows(BE, H),
        out_shape=jax.ShapeDtypeStruct((E, H), _f32),
    )(gr, gc, e2, w2, b2, wa, ba, wre)


def _tc_edge_equiv_body(gr_ref, gc_ref, e2_ref, w2_ref, b2_ref, w3_ref,
                        wre_ref, t_ref):
    v = gr_ref[...] + gc_ref[...] + _eterm(e2_ref[...], wre_ref[...])
    t1 = _silu(v)
    t2 = _silu(_dot(t1, w2_ref[...]) + b2_ref[...])
    t_ref[...] = lax.dot_general(w3_ref[...], t2, (((0,), (1,)), ((), ())),
                                 preferred_element_type=_f32)


def _edge_equiv(gr, gc, e2, w2, b2, w3, wre):
    return pl.pallas_call(
        _tc_edge_equiv_body,
        grid=(E // BE,),
        in_specs=[_rows(BE, H), _rows(BE, H),
                  pl.BlockSpec((2, BE), lambda i: (0, i)),
                  _full((H, H)), _full((1, H)), _full((H, 1)), _full((2, H))],
        out_specs=pl.BlockSpec((1, BE), lambda i: (0, i)),
        out_shape=jax.ShapeDtypeStruct((1, E), _f32),
    )(gr, gc, e2, w2, b2, w3, wre)


def _tc_trans_body(t_ref, cnx_ref, cny_ref, cnz_ref, tx_ref, ty_ref, tz_ref):
    t = t_ref[...]
    tx_ref[...] = cnx_ref[...] * t
    ty_ref[...] = cny_ref[...] * t
    tz_ref[...] = cnz_ref[...] * t


def _trans(t2d, cnx, cny, cnz):
    s = _full((ER, H))
    o = jax.ShapeDtypeStruct((ER, H), _f32)
    return pl.pallas_call(
        _tc_trans_body,
        grid=(1,),
        in_specs=[s] * 4,
        out_specs=[s] * 3,
        out_shape=[o] * 3,
    )(t2d, cnx, cny, cnz)


def _tc_node_body(h_ref, a0_ref, a1_ref, wna_ref, wnb_ref, bn1_ref, wn2_ref,
                  bn2_ref, wrn_ref, wcn_ref, brn_ref,
                  hout_ref, pr_ref, pc_ref):
    hb = h_ref[...]
    agg = a0_ref[...] + a1_ref[...]
    pre = _dot(hb, wna_ref[...]) + _dot(agg, wnb_ref[...]) + bn1_ref[...]
    n1 = _silu(pre)
    ho = hb + _dot(n1, wn2_ref[...]) + bn2_ref[...]
    hout_ref[...] = ho
    pr_ref[...] = _dot(ho, wrn_ref[...]) + brn_ref[...]
    pc_ref[...] = _dot(ho, wcn_ref[...])


def _node(h, a0, a1, wna, wnb, bn1, wn2, bn2, wrn, wcn, brn):
    return pl.pallas_call(
        _tc_node_body,
        grid=(N // BN,),
        in_specs=[_rows(BN, H), _rows(BN, H), _rows(BN, H),
                  _full((H, H)), _full((H, H)), _full((1, H)),
                  _full((H, H)), _full((1, H)),
                  _full((H, H)), _full((H, H)), _full((1, H))],
        out_specs=[_rows(BN, H)] * 3,
        out_shape=[jax.ShapeDtypeStruct((N, H), _f32)] * 3,
    )(h, a0, a1, wna, wnb, bn1, wn2, bn2, wrn, wcn, brn)


def _tc_final_body(x_ref, ax0, ax1, ay0, ay1, az0, az1, xout_ref):
    agg = jnp.concatenate([ax0[...] + ax1[...], ay0[...] + ay1[...],
                           az0[...] + az1[...]], axis=1)
    xout_ref[...] = x_ref[...] + agg * 0.01


def _final(x, ax0, ax1, ay0, ay1, az0, az1):
    return pl.pallas_call(
        _tc_final_body,
        grid=(N // BN,),
        in_specs=[_rows(BN, 3)] + [_rows(BN, 1)] * 6,
        out_specs=_rows(BN, 3),
        out_shape=jax.ShapeDtypeStruct((N, 3), _f32),
    )(x, ax0, ax1, ay0, ay1, az0, az1)


def _split_edge_w(lin):
    w = lin["w"]
    wre = jnp.stack([w[2 * H], w[2 * H + 1]], axis=0)
    return w[:H], w[H:2 * H], wre, lin["b"].reshape(1, H)


def kernel(h, x, edge_index, edge_attr, params):
    row = edge_index[0]
    col = edge_index[1]
    xx = x[:, 0]
    xy = x[:, 1]
    xz = x[:, 2]

    g0, g1, pe = params["gcl0"], params["gcl1"], params["equiv"]
    w1r0, w1c0, wre0, b10 = _split_edge_w(g0["edge1"])
    w1r1, w1c1, wre1, b11 = _split_edge_w(g1["edge1"])
    c1r, c1c, wree, bc1 = _split_edge_w(pe["c1"])

    zeros_h = jnp.zeros((N, H), _f32)

    def node_w(g):
        wn1 = g["node1"]["w"]
        return (wn1[:H], wn1[H:] * 0.01, g["node1"]["b"].reshape(1, H),
                g["node2"]["w"], g["node2"]["b"].reshape(1, H))

    pr0, pc0 = _proj(h, w1r0, w1c0, b10)
    gr0, gc0, oxr, oyr, ozr, oxc, oyc, ozc = _gather0()(
        pr0, pc0, xx, xy, xz, row, col)
    rad2, cnx2, cny2, cnz2 = _geom(
        oxr.reshape(ER, H), oyr.reshape(ER, H), ozr.reshape(ER, H),
        oxc.reshape(ER, H), oyc.reshape(ER, H), ozc.reshape(ER, H))
    e2 = jnp.stack([rad2.reshape(E), edge_attr.reshape(E)], axis=0)
    out0 = _edge_gcl(gr0, gc0, e2, g0["edge2"]["w"],
                     g0["edge2"]["b"].reshape(1, H), g0["att"]["w"],
                     g0["att"]["b"].reshape(1, 1), wre0)
    agg0 = _scatter_h(out0, row, zeros_h)
    wna, wnb, bn1, wn2, bn2 = node_w(g0)
    h1, pr1, pc1 = _node(h, agg0[0], agg0[1], wna, wnb, bn1, wn2, bn2,
                         w1r1, w1c1, b11)

    gr1, gc1 = _gather2()(pr1, pc1, row, col)
    out1 = _edge_gcl(gr1, gc1, e2, g1["edge2"]["w"],
                     g1["edge2"]["b"].reshape(1, H), g1["att"]["w"],
                     g1["att"]["b"].reshape(1, 1), wre1)
    agg1 = _scatter_h(out1, row, zeros_h)
    wna, wnb, bn1, wn2, bn2 = node_w(g1)
    h2, qr, qc = _node(h1, agg1[0], agg1[1], wna, wnb, bn1, wn2, bn2,
                       c1r, c1c, bc1)

    gqr, gqc = _gather2()(qr, qc, row, col)
    t1e = _edge_equiv(gqr, gqc, e2, pe["c2"]["w"],
                      pe["c2"]["b"].reshape(1, H), pe["c3w"], wree)
    tx2, ty2, tz2 = _trans(t1e.reshape(E).reshape(ER, H), cnx2, cny2, cnz2)
    zeros_1 = jnp.zeros((N,), _f32)
    ax, ay, az = _scatter3()(tx2.reshape(E), ty2.reshape(E), tz2.reshape(E),
                             row, zeros_1)
    x_out = _final(x,
                   ax[:N].reshape(N, 1), ax[N:].reshape(N, 1),
                   ay[:N].reshape(N, 1), ay[N:].reshape(N, 1),
                   az[:N].reshape(N, 1), az[N:].reshape(N, 1))

    return (h2, x_out)


# speedup vs baseline: 5.4193x; 1.0870x over previous
import functools

import jax
import jax.numpy as jnp
from jax import lax
from jax.experimental import pallas as pl
from jax.experimental.pallas import tpu as pltpu
import jax.experimental.pallas.tpu_sc as plsc

N = 10000
E = 320000
H = 128
ER = E // H

NC = 2
NS = 16
NW = NC * NS
EPW = E // NW
CH = 40
NCHUNK = EPW // CH
NSLOT = 5
NGRP = NCHUNK // NSLOT
EH = E // 2
EPW2 = EH // NW
NCHUNK2 = EPW2 // CH
NGRP2 = NCHUNK2 // NSLOT


def _mesh():
    return plsc.VectorSubcoreMesh(
        core_axis_name="c", subcore_axis_name="s",
        num_cores=NC, num_subcores=NS)


_f32 = jnp.float32


def _sc_gather0_body(pr, pc, xx, xy, xz, row, col,
                     gr, gc, oxr, oyr, ozr, oxc, oyc, ozc,
                     idx_r, idx_c, idx_rh, idx_ch, bufs, sems):
    wid = lax.axis_index("s") * NC + lax.axis_index("c")
    base = wid * EPW
    baseh = wid * EPW2
    pltpu.sync_copy(row.at[pl.ds(base, EPW)], idx_r)
    pltpu.sync_copy(col.at[pl.ds(base, EPW)], idx_c)
    pltpu.sync_copy(row.at[pl.ds(baseh, EPW2)], idx_rh)
    pltpu.sync_copy(col.at[pl.ds(baseh, EPW2)], idx_ch)

    bufr, bufc, bxr, byr, bzr, bxc, byc, bzc = bufs
    gsem, ssem = sems

    def fireh(i, s):
        pltpu.async_copy(pr.at[idx_rh.at[pl.ds(i * CH, CH)]], bufr.at[s], gsem[s])
        pltpu.async_copy(pc.at[idx_ch.at[pl.ds(i * CH, CH)]], bufc.at[s], gsem[s])

    for s in range(NSLOT):
        fireh(s, s)

    def grouph(g, carry):
        for s in range(NSLOT):
            i = g * NSLOT + s
            off = baseh + i * CH
            ir = idx_rh.at[pl.ds(i * CH, CH)]
            ic = idx_ch.at[pl.ds(i * CH, CH)]
            pltpu.make_async_copy(pr.at[ir], bufr.at[s], gsem[s]).wait()
            pltpu.make_async_copy(pc.at[ic], bufc.at[s], gsem[s]).wait()
            pltpu.async_copy(bufr.at[s], gr.at[pl.ds(off, CH)], ssem[s])
            pltpu.async_copy(bufc.at[s], gc.at[pl.ds(off, CH)], ssem[s])
            pltpu.make_async_copy(bufr.at[s], gr.at[pl.ds(off, CH)], ssem[s]).wait()
            pltpu.make_async_copy(bufc.at[s], gc.at[pl.ds(off, CH)], ssem[s]).wait()

            @pl.when(g < NGRP2 - 1)
            def _():
                fireh(i + NSLOT, s)
        return carry

    lax.fori_loop(0, NGRP2, grouph, 0)

    def gops(i, s):
        ir = idx_r.at[pl.ds(i * CH, CH)]
        ic = idx_c.at[pl.ds(i * CH, CH)]
        return [
            (xx.at[ir], bxr.at[s]), (xy.at[ir], byr.at[s]),
            (xz.at[ir], bzr.at[s]),
            (xx.at[ic], bxc.at[s]), (xy.at[ic], byc.at[s]),
            (xz.at[ic], bzc.at[s]),
        ]

    def sops(s, off):
        d = pl.ds(off, CH)
        return [
            (bxr.at[s], oxr.at[d]), (byr.at[s], oyr.at[d]),
            (bzr.at[s], ozr.at[d]),
            (bxc.at[s], oxc.at[d]), (byc.at[s], oyc.at[d]),
            (bzc.at[s], ozc.at[d]),
        ]

    def fire(i, s):
        for a, b in gops(i, s):
            pltpu.async_copy(a, b, gsem[s])

    for s in range(NSLOT):
        fire(s, s)

    def group(g, carry):
        for s in range(NSLOT):
            i = g * NSLOT + s
            off = base + i * CH
            for a, b in gops(i, s):
                pltpu.make_async_copy(a, b, gsem[s]).wait()
            for a, b in sops(s, off):
                pltpu.async_copy(a, b, ssem[s])
            for a, b in sops(s, off):
                pltpu.make_async_copy(a, b, ssem[s]).wait()

            @pl.when(g < NGRP - 1)
            def _():
                fire(i + NSLOT, s)
        return carry

    lax.fori_loop(0, NGRP, group, 0)


@functools.cache
def _gather0():
    vE = jax.ShapeDtypeStruct((E,), _f32)
    bC = pltpu.VMEM((NSLOT, CH), _f32)
    return pl.kernel(
        _sc_gather0_body,
        out_type=[
            jax.ShapeDtypeStruct((EH, H), _f32),
            jax.ShapeDtypeStruct((EH, H), _f32),
            vE, vE, vE, vE, vE, vE,
        ],
        mesh=_mesh(),
        scratch_types=[
            pltpu.VMEM((EPW,), jnp.int32),
            pltpu.VMEM((EPW,), jnp.int32),
            pltpu.VMEM((EPW2,), jnp.int32),
            pltpu.VMEM((EPW2,), jnp.int32),
            (pltpu.VMEM((NSLOT, CH, H), _f32), pltpu.VMEM((NSLOT, CH, H), _f32),
             bC, bC, bC, bC, bC, bC),
            ([pltpu.SemaphoreType.DMA] * NSLOT, [pltpu.SemaphoreType.DMA] * NSLOT),
        ],
    )


def _make_gather2_body(eoff):
    def body(pr, pc, row, col, gr, gc, idx_r, idx_c, bufs, sems):
        wid = lax.axis_index("s") * NC + lax.axis_index("c")
        base = eoff + wid * EPW2
        lbase = wid * EPW2
        pltpu.sync_copy(row.at[pl.ds(base, EPW2)], idx_r)
        pltpu.sync_copy(col.at[pl.ds(base, EPW2)], idx_c)

        bufr, bufc = bufs
        gsem, ssem = sems

        def fire(i, s):
            pltpu.async_copy(pr.at[idx_r.at[pl.ds(i * CH, CH)]], bufr.at[s],
                             gsem[s])
            pltpu.async_copy(pc.at[idx_c.at[pl.ds(i * CH, CH)]], bufc.at[s],
                             gsem[s])

        for s in range(NSLOT):
            fire(s, s)

        def group(g, carry):
            for s in range(NSLOT):
                i = g * NSLOT + s
                off = lbase + i * CH
                ir = idx_r.at[pl.ds(i * CH, CH)]
                ic = idx_c.at[pl.ds(i * CH, CH)]
                pltpu.make_async_copy(pr.at[ir], bufr.at[s], gsem[s]).wait()
                pltpu.make_async_copy(pc.at[ic], bufc.at[s], gsem[s]).wait()
                pltpu.async_copy(bufr.at[s], gr.at[pl.ds(off, CH)], ssem[s])
                pltpu.async_copy(bufc.at[s], gc.at[pl.ds(off, CH)], ssem[s])
                pltpu.make_async_copy(bufr.at[s], gr.at[pl.ds(off, CH)],
                                      ssem[s]).wait()
                pltpu.make_async_copy(bufc.at[s], gc.at[pl.ds(off, CH)],
                                      ssem[s]).wait()

                @pl.when(g < NGRP2 - 1)
                def _():
                    fire(i + NSLOT, s)
            return carry

        lax.fori_loop(0, NGRP2, group, 0)

    return body


@functools.cache
def _gather2h(eoff):
    return pl.kernel(
        _make_gather2_body(eoff),
        out_type=[
            jax.ShapeDtypeStruct((EH, H), _f32),
            jax.ShapeDtypeStruct((EH, H), _f32),
        ],
        mesh=_mesh(),
        scratch_types=[
            pltpu.VMEM((EPW2,), jnp.int32),
            pltpu.VMEM((EPW2,), jnp.int32),
            (pltpu.VMEM((NSLOT, CH, H), _f32), pltpu.VMEM((NSLOT, CH, H), _f32)),
            ([pltpu.SemaphoreType.DMA] * NSLOT, [pltpu.SemaphoreType.DMA] * NSLOT),
        ],
    )


@functools.cache
def _make_scatter(eoff):
    W = H

    def body(vals, row, zeros, agg, idx, buf, acc, sems):
        cid = lax.axis_index("c")
        sid = lax.axis_index("s")
        wid = cid * NS + sid
        base = eoff + wid * EPW2
        lbase = wid * EPW2
        gsem, ssem = sems

        @pl.when(sid == 0)
        def _():
            pltpu.sync_copy(zeros, acc)

        plsc.subcore_barrier()

        def fire(i, s):
            pltpu.async_copy(row.at[pl.ds(base + i * CH, CH)], idx.at[s], gsem[s])
            pltpu.async_copy(vals.at[pl.ds(lbase + i * CH, CH)], buf.at[s],
                             gsem[s])

        for s in range(NSLOT):
            fire(s, s)

        def group(g, carry):
            for s in range(NSLOT):
                i = g * NSLOT + s
                pltpu.make_async_copy(
                    row.at[pl.ds(base + i * CH, CH)], idx.at[s], gsem[s]).wait()
                pltpu.make_async_copy(
                    vals.at[pl.ds(lbase + i * CH, CH)], buf.at[s], gsem[s]).wait()
                cp = pltpu.async_copy(buf.at[s], acc.at[idx.at[s]], ssem[s],
                                      add=True)
                cp.wait()

                @pl.when(g < NGRP2 - 1)
                def _():
                    fire(i + NSLOT, s)
            return carry

        lax.fori_loop(0, NGRP2, group, 0)
        plsc.subcore_barrier()

        @pl.when(sid < NS - 1)
        def _():
            pltpu.sync_copy(acc.at[pl.ds(sid * 624, 624)],
                            agg.at[cid, pl.ds(sid * 624, 624)])

        @pl.when(sid == NS - 1)
        def _():
            pltpu.sync_copy(acc.at[pl.ds((NS - 1) * 624, N - (NS - 1) * 624)],
                            agg.at[cid, pl.ds((NS - 1) * 624, N - (NS - 1) * 624)])

    return pl.kernel(
        body,
        out_type=jax.ShapeDtypeStruct((NC, N, W), _f32),
        mesh=_mesh(),
        scratch_types=[
            pltpu.VMEM((NSLOT, CH), jnp.int32),
            pltpu.VMEM((NSLOT, CH, W), _f32),
            pltpu.VMEM_SHARED((N, W), _f32),
            ([pltpu.SemaphoreType.DMA] * NSLOT, [pltpu.SemaphoreType.DMA] * NSLOT),
        ],
    )


def _scatter_h(vals, row, zeros, eoff):
    return _make_scatter(eoff)(vals, row, zeros)


def _sc_scatter3_body(tx, ty, tz, row, zeros1, aggx, aggy, aggz,
                      idx, bufs, accx, accy, accz, bo, sems):
    cid = lax.axis_index("c")
    sid = lax.axis_index("s")
    wid = cid * NS + sid
    base = wid * EPW
    bx, by, bz = bufs
    gsem, ssem = sems

    @pl.when(sid == 0)
    def _():
        pltpu.sync_copy(zeros1, accx)
        pltpu.sync_copy(zeros1, accy)
        pltpu.sync_copy(zeros1, accz)

    plsc.subcore_barrier()

    def fire(i, s):
        off = base + i * CH
        pltpu.async_copy(row.at[pl.ds(off, CH)], idx.at[s], gsem[s])
        pltpu.async_copy(tx.at[pl.ds(off, CH)], bx.at[s], gsem[s])
        pltpu.async_copy(ty.at[pl.ds(off, CH)], by.at[s], gsem[s])
        pltpu.async_copy(tz.at[pl.ds(off, CH)], bz.at[s], gsem[s])

    for s in range(NSLOT):
        fire(s, s)

    def group(g, carry):
        for s in range(NSLOT):
            i = g * NSLOT + s
            off = base + i * CH
            pltpu.make_async_copy(row.at[pl.ds(off, CH)], idx.at[s], gsem[s]).wait()
            pltpu.make_async_copy(tx.at[pl.ds(off, CH)], bx.at[s], gsem[s]).wait()
            pltpu.make_async_copy(ty.at[pl.ds(off, CH)], by.at[s], gsem[s]).wait()
            pltpu.make_async_copy(tz.at[pl.ds(off, CH)], bz.at[s], gsem[s]).wait()
            pltpu.async_copy(bx.at[s], accx.at[idx.at[s]], ssem[s], add=True)
            pltpu.async_copy(by.at[s], accy.at[idx.at[s]], ssem[s], add=True)
            pltpu.async_copy(bz.at[s], accz.at[idx.at[s]], ssem[s], add=True)
            pltpu.make_async_copy(bx.at[s], accx.at[idx.at[s]], ssem[s]).wait()
            pltpu.make_async_copy(by.at[s], accy.at[idx.at[s]], ssem[s]).wait()
            pltpu.make_async_copy(bz.at[s], accz.at[idx.at[s]], ssem[s]).wait()

            @pl.when(g < NGRP - 1)
            def _():
                fire(i + NSLOT, s)
        return carry

    lax.fori_loop(0, NGRP, group, 0)
    plsc.subcore_barrier()

    sz0 = 624
    szL = N - (NS - 1) * sz0

    def wout(acc, agg, start, sz):
        pltpu.sync_copy(acc.at[pl.ds(start, sz)], bo.at[pl.ds(0, sz)])
        pltpu.sync_copy(bo.at[pl.ds(0, sz)], agg.at[pl.ds(cid * N + start, sz)])

    @pl.when(sid < NS - 1)
    def _():
        wout(accx, aggx, sid * sz0, sz0)
        wout(accy, aggy, sid * sz0, sz0)
        wout(accz, aggz, sid * sz0, sz0)

    @pl.when(sid == NS - 1)
    def _():
        wout(accx, aggx, (NS - 1) * sz0, szL)
        wout(accy, aggy, (NS - 1) * sz0, szL)
        wout(accz, aggz, (NS - 1) * sz0, szL)


@functools.cache
def _scatter3():
    vN = jax.ShapeDtypeStruct((NC * N,), _f32)
    bC = pltpu.VMEM((NSLOT, CH), _f32)
    aN = pltpu.VMEM_SHARED((N,), _f32)
    return pl.kernel(
        _sc_scatter3_body,
        out_type=[vN, vN, vN],
        mesh=_mesh(),
        scratch_types=[
            pltpu.VMEM((NSLOT, CH), jnp.int32),
            (bC, bC, bC),
            aN, aN, aN,
            pltpu.VMEM((640,), _f32),
            ([pltpu.SemaphoreType.DMA] * NSLOT, [pltpu.SemaphoreType.DMA] * NSLOT),
        ],
    )


BN = 2000
BE = 3200
BEL = BE // H


def _rows(bs, w):
    return pl.BlockSpec((bs, w), lambda i: (i, 0))


def _full(shape):
    return pl.BlockSpec(shape, lambda i: tuple(0 for _ in shape))


def _silu(v):
    return v * jax.nn.sigmoid(v)


def _dot(a, b):
    return jnp.dot(a, b, preferred_element_type=_f32)


def _tc_proj_body(h_ref, wr_ref, wc_ref, b_ref, pr_ref, pc_ref):
    hb = h_ref[...]
    pr_ref[...] = _dot(hb, wr_ref[...]) + b_ref[...]
    pc_ref[...] = _dot(hb, wc_ref[...])


def _proj(h, wr, wc, b):
    return pl.pallas_call(
        _tc_proj_body,
        grid=(N // BN,),
        in_specs=[_rows(BN, H), _full((H, H)), _full((H, H)), _full((1, H))],
        out_specs=[_rows(BN, H), _rows(BN, H)],
        out_shape=[jax.ShapeDtypeStruct((N, H), _f32)] * 2,
    )(h, wr, wc, b)


def _tc_geom_body(xr_ref, yr_ref, zr_ref, xc_ref, yc_ref, zc_ref,
                  rad_ref, cnx_ref, cny_ref, cnz_ref):
    cdx = xr_ref[...] - xc_ref[...]
    cdy = yr_ref[...] - yc_ref[...]
    cdz = zr_ref[...] - zc_ref[...]
    radial = cdx * cdx + cdy * cdy + cdz * cdz
    inv = 1.0 / (jnp.sqrt(radial + 1e-8) + 1.0)
    rad_ref[...] = radial
    cnx_ref[...] = cdx * inv
    cny_ref[...] = cdy * inv
    cnz_ref[...] = cdz * inv


def _geom(xr, yr, zr, xc, yc, zc):
    s = _full((ER, H))
    o = jax.ShapeDtypeStruct((ER, H), _f32)
    return pl.pallas_call(
        _tc_geom_body,
        grid=(1,),
        in_specs=[s] * 6,
        out_specs=[s] * 4,
        out_shape=[o] * 4,
    )(xr, yr, zr, xc, yc, zc)


def _eterm(e2blk, wre):
    return lax.dot_general(e2blk, wre, (((0,), (0,)), ((), ())),
                           preferred_element_type=_f32)


def _tc_edge_gcl_body(gr_ref, gc_ref, e2_ref, w2_ref, b2_ref, wa_ref, ba_ref,
                      wre_ref, out_ref):
    v = gr_ref[...] + gc_ref[...] + _eterm(e2_ref[...], wre_ref[...])
    m1 = _silu(v)
    mm = _dot(m1, w2_ref[...]) + b2_ref[...]
    m = _silu(mm)
    att = jax.nn.sigmoid(_dot(m, wa_ref[...]) + ba_ref[...])
    out_ref[...] = m * att


def _edge_gcl(gr, gc, e2, w2, b2, wa, ba, wre, half):
    nb = EH // BE
    return pl.pallas_call(
        _tc_edge_gcl_body,
        grid=(nb,),
        in_specs=[_rows(BE, H), _rows(BE, H),
                  pl.BlockSpec((2, BE), lambda i: (0, i + half * nb)),
                  _full((H, H)), _full((1, H)), _full((H, 1)), _full((1, 1)),
                  _full((2, H))],
        out_specs=_rows(BE, H),
        out_shape=jax.ShapeDtypeStruct((EH, H), _f32),
    )(gr, gc, e2, w2, b2, wa, ba, wre)


def _tc_edge_equiv_body(gr_ref, gc_ref, e2_ref, w2_ref, b2_ref, w3_ref,
                        wre_ref, t_ref):
    v = gr_ref[...] + gc_ref[...] + _eterm(e2_ref[...], wre_ref[...])
    t1 = _silu(v)
    t2 = _silu(_dot(t1, w2_ref[...]) + b2_ref[...])
    t_ref[...] = lax.dot_general(w3_ref[...], t2, (((0,), (1,)), ((), ())),
                                 preferred_element_type=_f32)


def _edge_equiv(gr, gc, e2, w2, b2, w3, wre, half):
    nb = EH // BE
    return pl.pallas_call(
        _tc_edge_equiv_body,
        grid=(nb,),
        in_specs=[_rows(BE, H), _rows(BE, H),
                  pl.BlockSpec((2, BE), lambda i: (0, i + half * nb)),
                  _full((H, H)), _full((1, H)), _full((H, 1)), _full((2, H))],
        out_specs=pl.BlockSpec((1, BE), lambda i: (0, i)),
        out_shape=jax.ShapeDtypeStruct((1, EH), _f32),
    )(gr, gc, e2, w2, b2, w3, wre)


def _tc_trans_body(t_ref, cnx_ref, cny_ref, cnz_ref, tx_ref, ty_ref, tz_ref):
    t = t_ref[...]
    tx_ref[...] = cnx_ref[...] * t
    ty_ref[...] = cny_ref[...] * t
    tz_ref[...] = cnz_ref[...] * t


def _trans(t2d, cnx, cny, cnz):
    s = _full((ER, H))
    o = jax.ShapeDtypeStruct((ER, H), _f32)
    return pl.pallas_call(
        _tc_trans_body,
        grid=(1,),
        in_specs=[s] * 4,
        out_specs=[s] * 3,
        out_shape=[o] * 3,
    )(t2d, cnx, cny, cnz)


def _tc_node_body(h_ref, a0_ref, a1_ref, a2_ref, a3_ref, wna_ref, wnb_ref,
                  bn1_ref, wn2_ref, bn2_ref, wrn_ref, wcn_ref, brn_ref,
                  hout_ref, pr_ref, pc_ref):
    hb = h_ref[...]
    agg = (a0_ref[...] + a1_ref[...]) + (a2_ref[...] + a3_ref[...])
    pre = _dot(hb, wna_ref[...]) + _dot(agg, wnb_ref[...]) + bn1_ref[...]
    n1 = _silu(pre)
    ho = hb + _dot(n1, wn2_ref[...]) + bn2_ref[...]
    hout_ref[...] = ho
    pr_ref[...] = _dot(ho, wrn_ref[...]) + brn_ref[...]
    pc_ref[...] = _dot(ho, wcn_ref[...])


def _node(h, a0, a1, a2, a3, wna, wnb, bn1, wn2, bn2, wrn, wcn, brn):
    return pl.pallas_call(
        _tc_node_body,
        grid=(N // BN,),
        in_specs=[_rows(BN, H)] * 5 +
                 [_full((H, H)), _full((H, H)), _full((1, H)),
                  _full((H, H)), _full((1, H)),
                  _full((H, H)), _full((H, H)), _full((1, H))],
        out_specs=[_rows(BN, H)] * 3,
        out_shape=[jax.ShapeDtypeStruct((N, H), _f32)] * 3,
    )(h, a0, a1, a2, a3, wna, wnb, bn1, wn2, bn2, wrn, wcn, brn)


def _tc_final_body(x_ref, ax0, ax1, ay0, ay1, az0, az1, xout_ref):
    agg = jnp.concatenate([ax0[...] + ax1[...], ay0[...] + ay1[...],
                           az0[...] + az1[...]], axis=1)
    xout_ref[...] = x_ref[...] + agg * 0.01


def _final(x, ax0, ax1, ay0, ay1, az0, az1):
    return pl.pallas_call(
        _tc_final_body,
        grid=(N // BN,),
        in_specs=[_rows(BN, 3)] + [_rows(BN, 1)] * 6,
        out_specs=_rows(BN, 3),
        out_shape=jax.ShapeDtypeStruct((N, 3), _f32),
    )(x, ax0, ax1, ay0, ay1, az0, az1)


def _split_edge_w(lin):
    w = lin["w"]
    wre = jnp.stack([w[2 * H], w[2 * H + 1]], axis=0)
    return w[:H], w[H:2 * H], wre, lin["b"].reshape(1, H)


def kernel(h, x, edge_index, edge_attr, params):
    row = edge_index[0]
    col = edge_index[1]
    xx = x[:, 0]
    xy = x[:, 1]
    xz = x[:, 2]

    g0, g1, pe = params["gcl0"], params["gcl1"], params["equiv"]
    w1r0, w1c0, wre0, b10 = _split_edge_w(g0["edge1"])
    w1r1, w1c1, wre1, b11 = _split_edge_w(g1["edge1"])
    c1r, c1c, wree, bc1 = _split_edge_w(pe["c1"])

    zeros_h = jnp.zeros((N, H), _f32)

    def node_w(g):
        wn1 = g["node1"]["w"]
        return (wn1[:H], wn1[H:] * 0.01, g["node1"]["b"].reshape(1, H),
                g["node2"]["w"], g["node2"]["b"].reshape(1, H))

    pr0, pc0 = _proj(h, w1r0, w1c0, b10)
    gr0a, gc0a, oxr, oyr, ozr, oxc, oyc, ozc = _gather0()(
        pr0, pc0, xx, xy, xz, row, col)
    gr0b, gc0b = _gather2h(EH)(pr0, pc0, row, col)
    rad2, cnx2, cny2, cnz2 = _geom(
        oxr.reshape(ER, H), oyr.reshape(ER, H), ozr.reshape(ER, H),
        oxc.reshape(ER, H), oyc.reshape(ER, H), ozc.reshape(ER, H))
    e2 = jnp.stack([rad2.reshape(E), edge_attr.reshape(E)], axis=0)
    w20 = g0["edge2"]["w"]
    b20 = g0["edge2"]["b"].reshape(1, H)
    wa0 = g0["att"]["w"]
    ba0 = g0["att"]["b"].reshape(1, 1)
    out0a = _edge_gcl(gr0a, gc0a, e2, w20, b20, wa0, ba0, wre0, 0)
    agg0a = _scatter_h(out0a, row, zeros_h, 0)
    out0b = _edge_gcl(gr0b, gc0b, e2, w20, b20, wa0, ba0, wre0, 1)
    agg0b = _scatter_h(out0b, row, zeros_h, EH)
    wna, wnb, bn1, wn2, bn2 = node_w(g0)
    h1, pr1, pc1 = _node(h, agg0a[0], agg0a[1], agg0b[0], agg0b[1],
                         wna, wnb, bn1, wn2, bn2, w1r1, w1c1, b11)

    gr1a, gc1a = _gather2h(0)(pr1, pc1, row, col)
    gr1b, gc1b = _gather2h(EH)(pr1, pc1, row, col)
    w21 = g1["edge2"]["w"]
    b21 = g1["edge2"]["b"].reshape(1, H)
    wa1 = g1["att"]["w"]
    ba1 = g1["att"]["b"].reshape(1, 1)
    out1a = _edge_gcl(gr1a, gc1a, e2, w21, b21, wa1, ba1, wre1, 0)
    agg1a = _scatter_h(out1a, row, zeros_h, 0)
    out1b = _edge_gcl(gr1b, gc1b, e2, w21, b21, wa1, ba1, wre1, 1)
    agg1b = _scatter_h(out1b, row, zeros_h, EH)
    wna, wnb, bn1, wn2, bn2 = node_w(g1)
    h2, qr, qc = _node(h1, agg1a[0], agg1a[1], agg1b[0], agg1b[1],
                       wna, wnb, bn1, wn2, bn2, c1r, c1c, bc1)

    gqra, gqca = _gather2h(0)(qr, qc, row, col)
    gqrb, gqcb = _gather2h(EH)(qr, qc, row, col)
    c2w = pe["c2"]["w"]
    c2b = pe["c2"]["b"].reshape(1, H)
    t1a = _edge_equiv(gqra, gqca, e2, c2w, c2b, pe["c3w"], wree, 0)
    t1b = _edge_equiv(gqrb, gqcb, e2, c2w, c2b, pe["c3w"], wree, 1)
    t1 = jnp.concatenate([t1a.reshape(EH), t1b.reshape(EH)])
    tx2, ty2, tz2 = _trans(t1.reshape(ER, H), cnx2, cny2, cnz2)
    zeros_1 = jnp.zeros((N,), _f32)
    ax, ay, az = _scatter3()(tx2.reshape(E), ty2.reshape(E), tz2.reshape(E),
                             row, zeros_1)
    x_out = _final(x,
                   ax[:N].reshape(N, 1), ax[N:].reshape(N, 1),
                   ay[:N].reshape(N, 1), ay[N:].reshape(N, 1),
                   az[:N].reshape(N, 1), az[N:].reshape(N, 1))

    return (h2, x_out)


# speedup vs baseline: 6.2988x; 1.1623x over previous
import functools

import jax
import jax.numpy as jnp
from jax import lax
from jax.experimental import pallas as pl
from jax.experimental.pallas import tpu as pltpu
import jax.experimental.pallas.tpu_sc as plsc

N = 10000
E = 320000
H = 128
ER = E // H

NC = 2
NS = 16
NW = NC * NS
EPW = E // NW
CH = 40
NCHUNK = EPW // CH
NSLOT = 5
NGRP = NCHUNK // NSLOT
EH = E // 2
EPW2 = EH // NW
NCHUNK2 = EPW2 // CH
NGRP2 = NCHUNK2 // NSLOT


def _mesh():
    return plsc.VectorSubcoreMesh(
        core_axis_name="c", subcore_axis_name="s",
        num_cores=NC, num_subcores=NS)


_f32 = jnp.float32


def _sc_gather0_body(pr, pc, xx, xy, xz, row, col,
                     gr, gc, oxr, oyr, ozr, oxc, oyc, ozc,
                     idx_r, idx_c, idx_rh, idx_ch, bufs, sems):
    wid = lax.axis_index("s") * NC + lax.axis_index("c")
    base = wid * EPW
    baseh = wid * EPW2
    pltpu.sync_copy(row.at[pl.ds(base, EPW)], idx_r)
    pltpu.sync_copy(col.at[pl.ds(base, EPW)], idx_c)
    pltpu.sync_copy(row.at[pl.ds(baseh, EPW2)], idx_rh)
    pltpu.sync_copy(col.at[pl.ds(baseh, EPW2)], idx_ch)

    bufr, bufc, bxr, byr, bzr, bxc, byc, bzc = bufs
    gsem, ssem = sems

    def fireh(i, s):
        pltpu.async_copy(pr.at[idx_rh.at[pl.ds(i * CH, CH)]], bufr.at[s], gsem[s])
        pltpu.async_copy(pc.at[idx_ch.at[pl.ds(i * CH, CH)]], bufc.at[s], gsem[s])

    for s in range(NSLOT):
        fireh(s, s)

    def grouph(g, carry):
        for s in range(NSLOT):
            i = g * NSLOT + s
            off = baseh + i * CH
            ir = idx_rh.at[pl.ds(i * CH, CH)]
            ic = idx_ch.at[pl.ds(i * CH, CH)]
            pltpu.make_async_copy(pr.at[ir], bufr.at[s], gsem[s]).wait()
            pltpu.make_async_copy(pc.at[ic], bufc.at[s], gsem[s]).wait()
            pltpu.async_copy(bufr.at[s], gr.at[pl.ds(off, CH)], ssem[s])
            pltpu.async_copy(bufc.at[s], gc.at[pl.ds(off, CH)], ssem[s])
            pltpu.make_async_copy(bufr.at[s], gr.at[pl.ds(off, CH)], ssem[s]).wait()
            pltpu.make_async_copy(bufc.at[s], gc.at[pl.ds(off, CH)], ssem[s]).wait()

            @pl.when(g < NGRP2 - 1)
            def _():
                fireh(i + NSLOT, s)
        return carry

    lax.fori_loop(0, NGRP2, grouph, 0)

    def gops(i, s):
        ir = idx_r.at[pl.ds(i * CH, CH)]
        ic = idx_c.at[pl.ds(i * CH, CH)]
        return [
            (xx.at[ir], bxr.at[s]), (xy.at[ir], byr.at[s]),
            (xz.at[ir], bzr.at[s]),
            (xx.at[ic], bxc.at[s]), (xy.at[ic], byc.at[s]),
            (xz.at[ic], bzc.at[s]),
        ]

    def sops(s, off):
        d = pl.ds(off, CH)
        return [
            (bxr.at[s], oxr.at[d]), (byr.at[s], oyr.at[d]),
            (bzr.at[s], ozr.at[d]),
            (bxc.at[s], oxc.at[d]), (byc.at[s], oyc.at[d]),
            (bzc.at[s], ozc.at[d]),
        ]

    def fire(i, s):
        for a, b in gops(i, s):
            pltpu.async_copy(a, b, gsem[s])

    for s in range(NSLOT):
        fire(s, s)

    def group(g, carry):
        for s in range(NSLOT):
            i = g * NSLOT + s
            off = base + i * CH
            for a, b in gops(i, s):
                pltpu.make_async_copy(a, b, gsem[s]).wait()
            for a, b in sops(s, off):
                pltpu.async_copy(a, b, ssem[s])
            for a, b in sops(s, off):
                pltpu.make_async_copy(a, b, ssem[s]).wait()

            @pl.when(g < NGRP - 1)
            def _():
                fire(i + NSLOT, s)
        return carry

    lax.fori_loop(0, NGRP, group, 0)


@functools.cache
def _gather0():
    vE = jax.ShapeDtypeStruct((E,), _f32)
    bC = pltpu.VMEM((NSLOT, CH), _f32)
    return pl.kernel(
        _sc_gather0_body,
        out_type=[
            jax.ShapeDtypeStruct((EH, H), _f32),
            jax.ShapeDtypeStruct((EH, H), _f32),
            vE, vE, vE, vE, vE, vE,
        ],
        mesh=_mesh(),
        scratch_types=[
            pltpu.VMEM((EPW,), jnp.int32),
            pltpu.VMEM((EPW,), jnp.int32),
            pltpu.VMEM((EPW2,), jnp.int32),
            pltpu.VMEM((EPW2,), jnp.int32),
            (pltpu.VMEM((NSLOT, CH, H), _f32), pltpu.VMEM((NSLOT, CH, H), _f32),
             bC, bC, bC, bC, bC, bC),
            ([pltpu.SemaphoreType.DMA] * NSLOT, [pltpu.SemaphoreType.DMA] * NSLOT),
        ],
    )


def _make_gather2_body(eoff):
    def body(pr, pc, row, col, gr, gc, idx_r, idx_c, bufs, sems):
        wid = lax.axis_index("s") * NC + lax.axis_index("c")
        base = eoff + wid * EPW2
        lbase = wid * EPW2
        pltpu.sync_copy(row.at[pl.ds(base, EPW2)], idx_r)
        pltpu.sync_copy(col.at[pl.ds(base, EPW2)], idx_c)

        bufr, bufc = bufs
        gsem, ssem = sems

        def fire(i, s):
            pltpu.async_copy(pr.at[idx_r.at[pl.ds(i * CH, CH)]], bufr.at[s],
                             gsem[s])
            pltpu.async_copy(pc.at[idx_c.at[pl.ds(i * CH, CH)]], bufc.at[s],
                             gsem[s])

        for s in range(NSLOT):
            fire(s, s)

        def group(g, carry):
            for s in range(NSLOT):
                i = g * NSLOT + s
                off = lbase + i * CH
                ir = idx_r.at[pl.ds(i * CH, CH)]
                ic = idx_c.at[pl.ds(i * CH, CH)]
                pltpu.make_async_copy(pr.at[ir], bufr.at[s], gsem[s]).wait()
                pltpu.make_async_copy(pc.at[ic], bufc.at[s], gsem[s]).wait()
                pltpu.async_copy(bufr.at[s], gr.at[pl.ds(off, CH)], ssem[s])
                pltpu.async_copy(bufc.at[s], gc.at[pl.ds(off, CH)], ssem[s])
                pltpu.make_async_copy(bufr.at[s], gr.at[pl.ds(off, CH)],
                                      ssem[s]).wait()
                pltpu.make_async_copy(bufc.at[s], gc.at[pl.ds(off, CH)],
                                      ssem[s]).wait()

                @pl.when(g < NGRP2 - 1)
                def _():
                    fire(i + NSLOT, s)
            return carry

        lax.fori_loop(0, NGRP2, group, 0)

    return body


@functools.cache
def _gather2h(eoff):
    return pl.kernel(
        _make_gather2_body(eoff),
        out_type=[
            jax.ShapeDtypeStruct((EH, H), _f32),
            jax.ShapeDtypeStruct((EH, H), _f32),
        ],
        mesh=_mesh(),
        scratch_types=[
            pltpu.VMEM((EPW2,), jnp.int32),
            pltpu.VMEM((EPW2,), jnp.int32),
            (pltpu.VMEM((NSLOT, CH, H), _f32), pltpu.VMEM((NSLOT, CH, H), _f32)),
            ([pltpu.SemaphoreType.DMA] * NSLOT, [pltpu.SemaphoreType.DMA] * NSLOT),
        ],
    )


def _make_gather2s_body(eoff):
    def body(pr, pc, row, col, gr, gc, idx, buf, acc, sems):
        cid = lax.axis_index("c")
        sid = lax.axis_index("s")
        gsem, ssem = sems

        @pl.when((sid == 0) & (cid == 0))
        def _():
            pltpu.sync_copy(pr, acc)

        @pl.when((sid == 0) & (cid == 1))
        def _():
            pltpu.sync_copy(pc, acc)

        plsc.subcore_barrier()

        def pipeline(idxarr, out):
            base = eoff + sid * (EH // NS)
            lbase = sid * (EH // NS)
            pltpu.sync_copy(idxarr.at[pl.ds(base, EPW)], idx)

            def fire(i, s):
                pltpu.async_copy(acc.at[idx.at[pl.ds(i * CH, CH)]], buf.at[s],
                                 gsem[s])

            for s in range(NSLOT):
                fire(s, s)

            def group(g, carry):
                for s in range(NSLOT):
                    i = g * NSLOT + s
                    off = lbase + i * CH
                    ii = idx.at[pl.ds(i * CH, CH)]
                    pltpu.make_async_copy(acc.at[ii], buf.at[s], gsem[s]).wait()
                    pltpu.async_copy(buf.at[s], out.at[pl.ds(off, CH)], ssem[s])
                    pltpu.make_async_copy(buf.at[s], out.at[pl.ds(off, CH)],
                                          ssem[s]).wait()

                    @pl.when(g < NGRP - 1)
                    def _():
                        fire(i + NSLOT, s)
                return carry

            lax.fori_loop(0, NGRP, group, 0)

        @pl.when(cid == 0)
        def _():
            pipeline(row, gr)

        @pl.when(cid == 1)
        def _():
            pipeline(col, gc)

    return body


@functools.cache
def _gather2s(eoff):
    return pl.kernel(
        _make_gather2s_body(eoff),
        out_type=[
            jax.ShapeDtypeStruct((EH, H), _f32),
            jax.ShapeDtypeStruct((EH, H), _f32),
        ],
        mesh=_mesh(),
        scratch_types=[
            pltpu.VMEM((EPW,), jnp.int32),
            pltpu.VMEM((NSLOT, CH, H), _f32),
            pltpu.VMEM_SHARED((N, H), _f32),
            ([pltpu.SemaphoreType.DMA] * NSLOT, [pltpu.SemaphoreType.DMA] * NSLOT),
        ],
    )


@functools.cache
def _make_scatter(eoff):
    W = H

    def body(vals, row, zeros, agg, idx, buf, acc, sems):
        cid = lax.axis_index("c")
        sid = lax.axis_index("s")
        wid = cid * NS + sid
        base = eoff + wid * EPW2
        lbase = wid * EPW2
        gsem, ssem = sems

        @pl.when(sid == 0)
        def _():
            pltpu.sync_copy(zeros, acc)

        plsc.subcore_barrier()

        def fire(i, s):
            pltpu.async_copy(row.at[pl.ds(base + i * CH, CH)], idx.at[s], gsem[s])
            pltpu.async_copy(vals.at[pl.ds(lbase + i * CH, CH)], buf.at[s],
                             gsem[s])

        for s in range(NSLOT):
            fire(s, s)

        def group(g, carry):
            for s in range(NSLOT):
                i = g * NSLOT + s
                pltpu.make_async_copy(
                    row.at[pl.ds(base + i * CH, CH)], idx.at[s], gsem[s]).wait()
                pltpu.make_async_copy(
                    vals.at[pl.ds(lbase + i * CH, CH)], buf.at[s], gsem[s]).wait()
                cp = pltpu.async_copy(buf.at[s], acc.at[idx.at[s]], ssem[s],
                                      add=True)
                cp.wait()

                @pl.when(g < NGRP2 - 1)
                def _():
                    fire(i + NSLOT, s)
            return carry

        lax.fori_loop(0, NGRP2, group, 0)
        plsc.subcore_barrier()

        @pl.when(sid < NS - 1)
        def _():
            pltpu.sync_copy(acc.at[pl.ds(sid * 624, 624)],
                            agg.at[cid, pl.ds(sid * 624, 624)])

        @pl.when(sid == NS - 1)
        def _():
            pltpu.sync_copy(acc.at[pl.ds((NS - 1) * 624, N - (NS - 1) * 624)],
                            agg.at[cid, pl.ds((NS - 1) * 624, N - (NS - 1) * 624)])

    return pl.kernel(
        body,
        out_type=jax.ShapeDtypeStruct((NC, N, W), _f32),
        mesh=_mesh(),
        scratch_types=[
            pltpu.VMEM((NSLOT, CH), jnp.int32),
            pltpu.VMEM((NSLOT, CH, W), _f32),
            pltpu.VMEM_SHARED((N, W), _f32),
            ([pltpu.SemaphoreType.DMA] * NSLOT, [pltpu.SemaphoreType.DMA] * NSLOT),
        ],
    )


def _scatter_h(vals, row, zeros, eoff):
    return _make_scatter(eoff)(vals, row, zeros)


def _sc_scatter3_body(tx, ty, tz, row, zeros1, aggx, aggy, aggz,
                      idx, bufs, accx, accy, accz, bo, sems):
    cid = lax.axis_index("c")
    sid = lax.axis_index("s")
    wid = cid * NS + sid
    base = wid * EPW
    bx, by, bz = bufs
    gsem, ssem = sems

    @pl.when(sid == 0)
    def _():
        pltpu.sync_copy(zeros1, accx)
        pltpu.sync_copy(zeros1, accy)
        pltpu.sync_copy(zeros1, accz)

    plsc.subcore_barrier()

    def fire(i, s):
        off = base + i * CH
        pltpu.async_copy(row.at[pl.ds(off, CH)], idx.at[s], gsem[s])
        pltpu.async_copy(tx.at[pl.ds(off, CH)], bx.at[s], gsem[s])
        pltpu.async_copy(ty.at[pl.ds(off, CH)], by.at[s], gsem[s])
        pltpu.async_copy(tz.at[pl.ds(off, CH)], bz.at[s], gsem[s])

    for s in range(NSLOT):
        fire(s, s)

    def group(g, carry):
        for s in range(NSLOT):
            i = g * NSLOT + s
            off = base + i * CH
            pltpu.make_async_copy(row.at[pl.ds(off, CH)], idx.at[s], gsem[s]).wait()
            pltpu.make_async_copy(tx.at[pl.ds(off, CH)], bx.at[s], gsem[s]).wait()
            pltpu.make_async_copy(ty.at[pl.ds(off, CH)], by.at[s], gsem[s]).wait()
            pltpu.make_async_copy(tz.at[pl.ds(off, CH)], bz.at[s], gsem[s]).wait()
            pltpu.async_copy(bx.at[s], accx.at[idx.at[s]], ssem[s], add=True)
            pltpu.async_copy(by.at[s], accy.at[idx.at[s]], ssem[s], add=True)
            pltpu.async_copy(bz.at[s], accz.at[idx.at[s]], ssem[s], add=True)
            pltpu.make_async_copy(bx.at[s], accx.at[idx.at[s]], ssem[s]).wait()
            pltpu.make_async_copy(by.at[s], accy.at[idx.at[s]], ssem[s]).wait()
            pltpu.make_async_copy(bz.at[s], accz.at[idx.at[s]], ssem[s]).wait()

            @pl.when(g < NGRP - 1)
            def _():
                fire(i + NSLOT, s)
        return carry

    lax.fori_loop(0, NGRP, group, 0)
    plsc.subcore_barrier()

    sz0 = 624
    szL = N - (NS - 1) * sz0

    def wout(acc, agg, start, sz):
        pltpu.sync_copy(acc.at[pl.ds(start, sz)], bo.at[pl.ds(0, sz)])
        pltpu.sync_copy(bo.at[pl.ds(0, sz)], agg.at[pl.ds(cid * N + start, sz)])

    @pl.when(sid < NS - 1)
    def _():
        wout(accx, aggx, sid * sz0, sz0)
        wout(accy, aggy, sid * sz0, sz0)
        wout(accz, aggz, sid * sz0, sz0)

    @pl.when(sid == NS - 1)
    def _():
        wout(accx, aggx, (NS - 1) * sz0, szL)
        wout(accy, aggy, (NS - 1) * sz0, szL)
        wout(accz, aggz, (NS - 1) * sz0, szL)


@functools.cache
def _scatter3():
    vN = jax.ShapeDtypeStruct((NC * N,), _f32)
    bC = pltpu.VMEM((NSLOT, CH), _f32)
    aN = pltpu.VMEM_SHARED((N,), _f32)
    return pl.kernel(
        _sc_scatter3_body,
        out_type=[vN, vN, vN],
        mesh=_mesh(),
        scratch_types=[
            pltpu.VMEM((NSLOT, CH), jnp.int32),
            (bC, bC, bC),
            aN, aN, aN,
            pltpu.VMEM((640,), _f32),
            ([pltpu.SemaphoreType.DMA] * NSLOT, [pltpu.SemaphoreType.DMA] * NSLOT),
        ],
    )


BN = 2000
BE = 3200
BEL = BE // H


def _rows(bs, w):
    return pl.BlockSpec((bs, w), lambda i: (i, 0))


def _full(shape):
    return pl.BlockSpec(shape, lambda i: tuple(0 for _ in shape))


def _silu(v):
    return v * jax.nn.sigmoid(v)


def _dot(a, b):
    return jnp.dot(a, b, preferred_element_type=_f32)


def _tc_proj_body(h_ref, wr_ref, wc_ref, b_ref, pr_ref, pc_ref):
    hb = h_ref[...]
    pr_ref[...] = _dot(hb, wr_ref[...]) + b_ref[...]
    pc_ref[...] = _dot(hb, wc_ref[...])


def _proj(h, wr, wc, b):
    return pl.pallas_call(
        _tc_proj_body,
        grid=(N // BN,),
        in_specs=[_rows(BN, H), _full((H, H)), _full((H, H)), _full((1, H))],
        out_specs=[_rows(BN, H), _rows(BN, H)],
        out_shape=[jax.ShapeDtypeStruct((N, H), _f32)] * 2,
    )(h, wr, wc, b)


def _tc_geom_body(xr_ref, yr_ref, zr_ref, xc_ref, yc_ref, zc_ref,
                  rad_ref, cnx_ref, cny_ref, cnz_ref):
    cdx = xr_ref[...] - xc_ref[...]
    cdy = yr_ref[...] - yc_ref[...]
    cdz = zr_ref[...] - zc_ref[...]
    radial = cdx * cdx + cdy * cdy + cdz * cdz
    inv = 1.0 / (jnp.sqrt(radial + 1e-8) + 1.0)
    rad_ref[...] = radial
    cnx_ref[...] = cdx * inv
    cny_ref[...] = cdy * inv
    cnz_ref[...] = cdz * inv


def _geom(xr, yr, zr, xc, yc, zc):
    s = _full((ER, H))
    o = jax.ShapeDtypeStruct((ER, H), _f32)
    return pl.pallas_call(
        _tc_geom_body,
        grid=(1,),
        in_specs=[s] * 6,
        out_specs=[s] * 4,
        out_shape=[o] * 4,
    )(xr, yr, zr, xc, yc, zc)


def _eterm(e2blk, wre):
    return lax.dot_general(e2blk, wre, (((0,), (0,)), ((), ())),
                           preferred_element_type=_f32)


def _tc_edge_gcl_body(gr_ref, gc_ref, e2_ref, w2_ref, b2_ref, wa_ref, ba_ref,
                      wre_ref, out_ref):
    v = gr_ref[...] + gc_ref[...] + _eterm(e2_ref[...], wre_ref[...])
    m1 = _silu(v)
    mm = _dot(m1, w2_ref[...]) + b2_ref[...]
    m = _silu(mm)
    att = jax.nn.sigmoid(_dot(m, wa_ref[...]) + ba_ref[...])
    out_ref[...] = m * att


def _edge_gcl(gr, gc, e2, w2, b2, wa, ba, wre, half):
    nb = EH // BE
    return pl.pallas_call(
        _tc_edge_gcl_body,
        grid=(nb,),
        in_specs=[_rows(BE, H), _rows(BE, H),
                  pl.BlockSpec((2, BE), lambda i: (0, i + half * nb)),
                  _full((H, H)), _full((1, H)), _full((H, 1)), _full((1, 1)),
                  _full((2, H))],
        out_specs=_rows(BE, H),
        out_shape=jax.ShapeDtypeStruct((EH, H), _f32),
    )(gr, gc, e2, w2, b2, wa, ba, wre)


def _tc_edge_equiv_body(gr_ref, gc_ref, e2_ref, w2_ref, b2_ref, w3_ref,
                        wre_ref, t_ref):
    v = gr_ref[...] + gc_ref[...] + _eterm(e2_ref[...], wre_ref[...])
    t1 = _silu(v)
    t2 = _silu(_dot(t1, w2_ref[...]) + b2_ref[...])
    t_ref[...] = lax.dot_general(w3_ref[...], t2, (((0,), (1,)), ((), ())),
                                 preferred_element_type=_f32)


def _edge_equiv(gr, gc, e2, w2, b2, w3, wre, half):
    nb = EH // BE
    return pl.pallas_call(
        _tc_edge_equiv_body,
        grid=(nb,),
        in_specs=[_rows(BE, H), _rows(BE, H),
                  pl.BlockSpec((2, BE), lambda i: (0, i + half * nb)),
                  _full((H, H)), _full((1, H)), _full((H, 1)), _full((2, H))],
        out_specs=pl.BlockSpec((1, BE), lambda i: (0, i)),
        out_shape=jax.ShapeDtypeStruct((1, EH), _f32),
    )(gr, gc, e2, w2, b2, w3, wre)


def _tc_trans_body(t_ref, cnx_ref, cny_ref, cnz_ref, tx_ref, ty_ref, tz_ref):
    t = t_ref[...]
    tx_ref[...] = cnx_ref[...] * t
    ty_ref[...] = cny_ref[...] * t
    tz_ref[...] = cnz_ref[...] * t


def _trans(t2d, cnx, cny, cnz):
    s = _full((ER, H))
    o = jax.ShapeDtypeStruct((ER, H), _f32)
    return pl.pallas_call(
        _tc_trans_body,
        grid=(1,),
        in_specs=[s] * 4,
        out_specs=[s] * 3,
        out_shape=[o] * 3,
    )(t2d, cnx, cny, cnz)


def _tc_node_body(h_ref, a0_ref, a1_ref, a2_ref, a3_ref, wna_ref, wnb_ref,
                  bn1_ref, wn2_ref, bn2_ref, wrn_ref, wcn_ref, brn_ref,
                  hout_ref, pr_ref, pc_ref):
    hb = h_ref[...]
    agg = (a0_ref[...] + a1_ref[...]) + (a2_ref[...] + a3_ref[...])
    pre = _dot(hb, wna_ref[...]) + _dot(agg, wnb_ref[...]) + bn1_ref[...]
    n1 = _silu(pre)
    ho = hb + _dot(n1, wn2_ref[...]) + bn2_ref[...]
    hout_ref[...] = ho
    pr_ref[...] = _dot(ho, wrn_ref[...]) + brn_ref[...]
    pc_ref[...] = _dot(ho, wcn_ref[...])


def _node(h, a0, a1, a2, a3, wna, wnb, bn1, wn2, bn2, wrn, wcn, brn):
    return pl.pallas_call(
        _tc_node_body,
        grid=(N // BN,),
        in_specs=[_rows(BN, H)] * 5 +
                 [_full((H, H)), _full((H, H)), _full((1, H)),
                  _full((H, H)), _full((1, H)),
                  _full((H, H)), _full((H, H)), _full((1, H))],
        out_specs=[_rows(BN, H)] * 3,
        out_shape=[jax.ShapeDtypeStruct((N, H), _f32)] * 3,
    )(h, a0, a1, a2, a3, wna, wnb, bn1, wn2, bn2, wrn, wcn, brn)


def _tc_final_body(x_ref, ax0, ax1, ay0, ay1, az0, az1, xout_ref):
    agg = jnp.concatenate([ax0[...] + ax1[...], ay0[...] + ay1[...],
                           az0[...] + az1[...]], axis=1)
    xout_ref[...] = x_ref[...] + agg * 0.01


def _final(x, ax0, ax1, ay0, ay1, az0, az1):
    return pl.pallas_call(
        _tc_final_body,
        grid=(N // BN,),
        in_specs=[_rows(BN, 3)] + [_rows(BN, 1)] * 6,
        out_specs=_rows(BN, 3),
        out_shape=jax.ShapeDtypeStruct((N, 3), _f32),
    )(x, ax0, ax1, ay0, ay1, az0, az1)


def _split_edge_w(lin):
    w = lin["w"]
    wre = jnp.stack([w[2 * H], w[2 * H + 1]], axis=0)
    return w[:H], w[H:2 * H], wre, lin["b"].reshape(1, H)


def kernel(h, x, edge_index, edge_attr, params):
    row = edge_index[0]
    col = edge_index[1]
    xx = x[:, 0]
    xy = x[:, 1]
    xz = x[:, 2]

    g0, g1, pe = params["gcl0"], params["gcl1"], params["equiv"]
    w1r0, w1c0, wre0, b10 = _split_edge_w(g0["edge1"])
    w1r1, w1c1, wre1, b11 = _split_edge_w(g1["edge1"])
    c1r, c1c, wree, bc1 = _split_edge_w(pe["c1"])

    zeros_h = jnp.zeros((N, H), _f32)

    def node_w(g):
        wn1 = g["node1"]["w"]
        return (wn1[:H], wn1[H:] * 0.01, g["node1"]["b"].reshape(1, H),
                g["node2"]["w"], g["node2"]["b"].reshape(1, H))

    pr0, pc0 = _proj(h, w1r0, w1c0, b10)
    gr0a, gc0a, oxr, oyr, ozr, oxc, oyc, ozc = _gather0()(
        pr0, pc0, xx, xy, xz, row, col)
    gr0b, gc0b = _gather2s(EH)(pr0, pc0, row, col)
    rad2, cnx2, cny2, cnz2 = _geom(
        oxr.reshape(ER, H), oyr.reshape(ER, H), ozr.reshape(ER, H),
        oxc.reshape(ER, H), oyc.reshape(ER, H), ozc.reshape(ER, H))
    e2 = jnp.stack([rad2.reshape(E), edge_attr.reshape(E)], axis=0)
    w20 = g0["edge2"]["w"]
    b20 = g0["edge2"]["b"].reshape(1, H)
    wa0 = g0["att"]["w"]
    ba0 = g0["att"]["b"].reshape(1, 1)
    out0a = _edge_gcl(gr0a, gc0a, e2, w20, b20, wa0, ba0, wre0, 0)
    agg0a = _scatter_h(out0a, row, zeros_h, 0)
    out0b = _edge_gcl(gr0b, gc0b, e2, w20, b20, wa0, ba0, wre0, 1)
    agg0b = _scatter_h(out0b, row, zeros_h, EH)
    wna, wnb, bn1, wn2, bn2 = node_w(g0)
    h1, pr1, pc1 = _node(h, agg0a[0], agg0a[1], agg0b[0], agg0b[1],
                         wna, wnb, bn1, wn2, bn2, w1r1, w1c1, b11)

    gr1a, gc1a = _gather2s(0)(pr1, pc1, row, col)
    gr1b, gc1b = _gather2s(EH)(pr1, pc1, row, col)
    w21 = g1["edge2"]["w"]
    b21 = g1["edge2"]["b"].reshape(1, H)
    wa1 = g1["att"]["w"]
    ba1 = g1["att"]["b"].reshape(1, 1)
    out1a = _edge_gcl(gr1a, gc1a, e2, w21, b21, wa1, ba1, wre1, 0)
    agg1a = _scatter_h(out1a, row, zeros_h, 0)
    out1b = _edge_gcl(gr1b, gc1b, e2, w21, b21, wa1, ba1, wre1, 1)
    agg1b = _scatter_h(out1b, row, zeros_h, EH)
    wna, wnb, bn1, wn2, bn2 = node_w(g1)
    h2, qr, qc = _node(h1, agg1a[0], agg1a[1], agg1b[0], agg1b[1],
                       wna, wnb, bn1, wn2, bn2, c1r, c1c, bc1)

    gqra, gqca = _gather2s(0)(qr, qc, row, col)
    gqrb, gqcb = _gather2s(EH)(qr, qc, row, col)
    c2w = pe["c2"]["w"]
    c2b = pe["c2"]["b"].reshape(1, H)
    t1a = _edge_equiv(gqra, gqca, e2, c2w, c2b, pe["c3w"], wree, 0)
    t1b = _edge_equiv(gqrb, gqcb, e2, c2w, c2b, pe["c3w"], wree, 1)
    t1 = jnp.concatenate([t1a.reshape(EH), t1b.reshape(EH)])
    tx2, ty2, tz2 = _trans(t1.reshape(ER, H), cnx2, cny2, cnz2)
    zeros_1 = jnp.zeros((N,), _f32)
    ax, ay, az = _scatter3()(tx2.reshape(E), ty2.reshape(E), tz2.reshape(E),
                             row, zeros_1)
    x_out = _final(x,
                   ax[:N].reshape(N, 1), ax[N:].reshape(N, 1),
                   ay[:N].reshape(N, 1), ay[N:].reshape(N, 1),
                   az[:N].reshape(N, 1), az[N:].reshape(N, 1))

    return (h2, x_out)


# speedup vs baseline: 6.5567x; 1.0409x over previous
import functools

import jax
import jax.numpy as jnp
from jax import lax
from jax.experimental import pallas as pl
from jax.experimental.pallas import tpu as pltpu
import jax.experimental.pallas.tpu_sc as plsc

N = 10000
E = 320000
H = 128
ER = E // H

NC = 2
NS = 16
NW = NC * NS
EPW = E // NW
CH = 40
NCHUNK = EPW // CH
NSLOT = 5
NGRP = NCHUNK // NSLOT
EH = E // 2
EPW2 = EH // NW
NCHUNK2 = EPW2 // CH
NGRP2 = NCHUNK2 // NSLOT


def _mesh():
    return plsc.VectorSubcoreMesh(
        core_axis_name="c", subcore_axis_name="s",
        num_cores=NC, num_subcores=NS)


_f32 = jnp.float32


def _sc_gather0_body(xx, xy, xz, row, col,
                     oxr, oyr, ozr, oxc, oyc, ozc,
                     idx_r, idx_c, bufs, sems):
    wid = lax.axis_index("s") * NC + lax.axis_index("c")
    base = wid * EPW
    pltpu.sync_copy(row.at[pl.ds(base, EPW)], idx_r)
    pltpu.sync_copy(col.at[pl.ds(base, EPW)], idx_c)

    bxr, byr, bzr, bxc, byc, bzc = bufs
    gsem, ssem = sems

    def gops(i, s):
        ir = idx_r.at[pl.ds(i * CH, CH)]
        ic = idx_c.at[pl.ds(i * CH, CH)]
        return [
            (xx.at[ir], bxr.at[s]), (xy.at[ir], byr.at[s]),
            (xz.at[ir], bzr.at[s]),
            (xx.at[ic], bxc.at[s]), (xy.at[ic], byc.at[s]),
            (xz.at[ic], bzc.at[s]),
        ]

    def sops(s, off):
        d = pl.ds(off, CH)
        return [
            (bxr.at[s], oxr.at[d]), (byr.at[s], oyr.at[d]),
            (bzr.at[s], ozr.at[d]),
            (bxc.at[s], oxc.at[d]), (byc.at[s], oyc.at[d]),
            (bzc.at[s], ozc.at[d]),
        ]

    def fire(i, s):
        for a, b in gops(i, s):
            pltpu.async_copy(a, b, gsem[s])

    for s in range(NSLOT):
        fire(s, s)

    def group(g, carry):
        for s in range(NSLOT):
            i = g * NSLOT + s
            off = base + i * CH
            for a, b in gops(i, s):
                pltpu.make_async_copy(a, b, gsem[s]).wait()
            for a, b in sops(s, off):
                pltpu.async_copy(a, b, ssem[s])
            for a, b in sops(s, off):
                pltpu.make_async_copy(a, b, ssem[s]).wait()

            @pl.when(g < NGRP - 1)
            def _():
                fire(i + NSLOT, s)
        return carry

    lax.fori_loop(0, NGRP, group, 0)


@functools.cache
def _gather0():
    vE = jax.ShapeDtypeStruct((E,), _f32)
    bC = pltpu.VMEM((NSLOT, CH), _f32)
    return pl.kernel(
        _sc_gather0_body,
        out_type=[vE, vE, vE, vE, vE, vE],
        mesh=_mesh(),
        scratch_types=[
            pltpu.VMEM((EPW,), jnp.int32),
            pltpu.VMEM((EPW,), jnp.int32),
            (bC, bC, bC, bC, bC, bC),
            ([pltpu.SemaphoreType.DMA] * NSLOT, [pltpu.SemaphoreType.DMA] * NSLOT),
        ],
    )


def _make_gather2_body(eoff):
    def body(pr, pc, row, col, gr, gc, idx_r, idx_c, bufs, sems):
        wid = lax.axis_index("s") * NC + lax.axis_index("c")
        base = eoff + wid * EPW2
        lbase = wid * EPW2
        pltpu.sync_copy(row.at[pl.ds(base, EPW2)], idx_r)
        pltpu.sync_copy(col.at[pl.ds(base, EPW2)], idx_c)

        bufr, bufc = bufs
        gsem, ssem = sems

        def fire(i, s):
            pltpu.async_copy(pr.at[idx_r.at[pl.ds(i * CH, CH)]], bufr.at[s],
                             gsem[s])
            pltpu.async_copy(pc.at[idx_c.at[pl.ds(i * CH, CH)]], bufc.at[s],
                             gsem[s])

        for s in range(NSLOT):
            fire(s, s)

        def group(g, carry):
            for s in range(NSLOT):
                i = g * NSLOT + s
                off = lbase + i * CH
                ir = idx_r.at[pl.ds(i * CH, CH)]
                ic = idx_c.at[pl.ds(i * CH, CH)]
                pltpu.make_async_copy(pr.at[ir], bufr.at[s], gsem[s]).wait()
                pltpu.make_async_copy(pc.at[ic], bufc.at[s], gsem[s]).wait()
                pltpu.async_copy(bufr.at[s], gr.at[pl.ds(off, CH)], ssem[s])
                pltpu.async_copy(bufc.at[s], gc.at[pl.ds(off, CH)], ssem[s])
                pltpu.make_async_copy(bufr.at[s], gr.at[pl.ds(off, CH)],
                                      ssem[s]).wait()
                pltpu.make_async_copy(bufc.at[s], gc.at[pl.ds(off, CH)],
                                      ssem[s]).wait()

                @pl.when(g < NGRP2 - 1)
                def _():
                    fire(i + NSLOT, s)
            return carry

        lax.fori_loop(0, NGRP2, group, 0)

    return body


@functools.cache
def _gather2h(eoff):
    return pl.kernel(
        _make_gather2_body(eoff),
        out_type=[
            jax.ShapeDtypeStruct((EH, H), _f32),
            jax.ShapeDtypeStruct((EH, H), _f32),
        ],
        mesh=_mesh(),
        scratch_types=[
            pltpu.VMEM((EPW2,), jnp.int32),
            pltpu.VMEM((EPW2,), jnp.int32),
            (pltpu.VMEM((NSLOT, CH, H), _f32), pltpu.VMEM((NSLOT, CH, H), _f32)),
            ([pltpu.SemaphoreType.DMA] * NSLOT, [pltpu.SemaphoreType.DMA] * NSLOT),
        ],
    )


def _make_gather2s_body(eoff):
    def body(pr, pc, row, col, gr, gc, idx, buf, acc, sems):
        cid = lax.axis_index("c")
        sid = lax.axis_index("s")
        gsem, ssem = sems

        @pl.when((sid == 0) & (cid == 0))
        def _():
            pltpu.sync_copy(pr, acc)

        @pl.when((sid == 0) & (cid == 1))
        def _():
            pltpu.sync_copy(pc, acc)

        plsc.subcore_barrier()

        def pipeline(idxarr, out):
            base = eoff + sid * (EH // NS)
            lbase = sid * (EH // NS)
            pltpu.sync_copy(idxarr.at[pl.ds(base, EPW)], idx)

            def fire(i, s):
                pltpu.async_copy(acc.at[idx.at[pl.ds(i * CH, CH)]], buf.at[s],
                                 gsem[s])

            for s in range(NSLOT):
                fire(s, s)

            def group(g, carry):
                for s in range(NSLOT):
                    i = g * NSLOT + s
                    off = lbase + i * CH
                    ii = idx.at[pl.ds(i * CH, CH)]
                    pltpu.make_async_copy(acc.at[ii], buf.at[s], gsem[s]).wait()
                    pltpu.async_copy(buf.at[s], out.at[pl.ds(off, CH)], ssem[s])
                    pltpu.make_async_copy(buf.at[s], out.at[pl.ds(off, CH)],
                                          ssem[s]).wait()

                    @pl.when(g < NGRP - 1)
                    def _():
                        fire(i + NSLOT, s)
                return carry

            lax.fori_loop(0, NGRP, group, 0)

        @pl.when(cid == 0)
        def _():
            pipeline(row, gr)

        @pl.when(cid == 1)
        def _():
            pipeline(col, gc)

    return body


@functools.cache
def _gather2s(eoff):
    return pl.kernel(
        _make_gather2s_body(eoff),
        out_type=[
            jax.ShapeDtypeStruct((EH, H), _f32),
            jax.ShapeDtypeStruct((EH, H), _f32),
        ],
        mesh=_mesh(),
        scratch_types=[
            pltpu.VMEM((EPW,), jnp.int32),
            pltpu.VMEM((NSLOT, CH, H), _f32),
            pltpu.VMEM_SHARED((N, H), _f32),
            ([pltpu.SemaphoreType.DMA] * NSLOT, [pltpu.SemaphoreType.DMA] * NSLOT),
        ],
    )


@functools.cache
def _make_scatter(eoff):
    W = H

    def body(vals, row, zeros, agg, idx, buf, acc, sems):
        cid = lax.axis_index("c")
        sid = lax.axis_index("s")
        wid = cid * NS + sid
        base = eoff + wid * EPW2
        lbase = wid * EPW2
        gsem, ssem = sems

        @pl.when(sid == 0)
        def _():
            pltpu.sync_copy(zeros, acc)

        plsc.subcore_barrier()

        def fire(i, s):
            pltpu.async_copy(row.at[pl.ds(base + i * CH, CH)], idx.at[s], gsem[s])
            pltpu.async_copy(vals.at[pl.ds(lbase + i * CH, CH)], buf.at[s],
                             gsem[s])

        for s in range(NSLOT):
            fire(s, s)

        def group(g, carry):
            for s in range(NSLOT):
                i = g * NSLOT + s
                pltpu.make_async_copy(
                    row.at[pl.ds(base + i * CH, CH)], idx.at[s], gsem[s]).wait()
                pltpu.make_async_copy(
                    vals.at[pl.ds(lbase + i * CH, CH)], buf.at[s], gsem[s]).wait()
                cp = pltpu.async_copy(buf.at[s], acc.at[idx.at[s]], ssem[s],
                                      add=True)
                cp.wait()

                @pl.when(g < NGRP2 - 1)
                def _():
                    fire(i + NSLOT, s)
            return carry

        lax.fori_loop(0, NGRP2, group, 0)
        plsc.subcore_barrier()

        @pl.when(sid < NS - 1)
        def _():
            pltpu.sync_copy(acc.at[pl.ds(sid * 624, 624)],
                            agg.at[cid, pl.ds(sid * 624, 624)])

        @pl.when(sid == NS - 1)
        def _():
            pltpu.sync_copy(acc.at[pl.ds((NS - 1) * 624, N - (NS - 1) * 624)],
                            agg.at[cid, pl.ds((NS - 1) * 624, N - (NS - 1) * 624)])

    return pl.kernel(
        body,
        out_type=jax.ShapeDtypeStruct((NC, N, W), _f32),
        mesh=_mesh(),
        scratch_types=[
            pltpu.VMEM((NSLOT, CH), jnp.int32),
            pltpu.VMEM((NSLOT, CH, W), _f32),
            pltpu.VMEM_SHARED((N, W), _f32),
            ([pltpu.SemaphoreType.DMA] * NSLOT, [pltpu.SemaphoreType.DMA] * NSLOT),
        ],
    )


def _scatter_h(vals, row, zeros, eoff):
    return _make_scatter(eoff)(vals, row, zeros)


def _sc_scatter3_body(tx, ty, tz, row, zeros1, aggx, aggy, aggz,
                      idx, bufs, accx, accy, accz, bo, sems):
    cid = lax.axis_index("c")
    sid = lax.axis_index("s")
    wid = cid * NS + sid
    base = wid * EPW
    bx, by, bz = bufs
    gsem, ssem = sems

    @pl.when(sid == 0)
    def _():
        pltpu.sync_copy(zeros1, accx)
        pltpu.sync_copy(zeros1, accy)
        pltpu.sync_copy(zeros1, accz)

    plsc.subcore_barrier()

    def fire(i, s):
        off = base + i * CH
        pltpu.async_copy(row.at[pl.ds(off, CH)], idx.at[s], gsem[s])
        pltpu.async_copy(tx.at[pl.ds(off, CH)], bx.at[s], gsem[s])
        pltpu.async_copy(ty.at[pl.ds(off, CH)], by.at[s], gsem[s])
        pltpu.async_copy(tz.at[pl.ds(off, CH)], bz.at[s], gsem[s])

    for s in range(NSLOT):
        fire(s, s)

    def group(g, carry):
        for s in range(NSLOT):
            i = g * NSLOT + s
            off = base + i * CH
            pltpu.make_async_copy(row.at[pl.ds(off, CH)], idx.at[s], gsem[s]).wait()
            pltpu.make_async_copy(tx.at[pl.ds(off, CH)], bx.at[s], gsem[s]).wait()
            pltpu.make_async_copy(ty.at[pl.ds(off, CH)], by.at[s], gsem[s]).wait()
            pltpu.make_async_copy(tz.at[pl.ds(off, CH)], bz.at[s], gsem[s]).wait()
            pltpu.async_copy(bx.at[s], accx.at[idx.at[s]], ssem[s], add=True)
            pltpu.async_copy(by.at[s], accy.at[idx.at[s]], ssem[s], add=True)
            pltpu.async_copy(bz.at[s], accz.at[idx.at[s]], ssem[s], add=True)
            pltpu.make_async_copy(bx.at[s], accx.at[idx.at[s]], ssem[s]).wait()
            pltpu.make_async_copy(by.at[s], accy.at[idx.at[s]], ssem[s]).wait()
            pltpu.make_async_copy(bz.at[s], accz.at[idx.at[s]], ssem[s]).wait()

            @pl.when(g < NGRP - 1)
            def _():
                fire(i + NSLOT, s)
        return carry

    lax.fori_loop(0, NGRP, group, 0)
    plsc.subcore_barrier()

    sz0 = 624
    szL = N - (NS - 1) * sz0

    def wout(acc, agg, start, sz):
        pltpu.sync_copy(acc.at[pl.ds(start, sz)], bo.at[pl.ds(0, sz)])
        pltpu.sync_copy(bo.at[pl.ds(0, sz)], agg.at[pl.ds(cid * N + start, sz)])

    @pl.when(sid < NS - 1)
    def _():
        wout(accx, aggx, sid * sz0, sz0)
        wout(accy, aggy, sid * sz0, sz0)
        wout(accz, aggz, sid * sz0, sz0)

    @pl.when(sid == NS - 1)
    def _():
        wout(accx, aggx, (NS - 1) * sz0, szL)
        wout(accy, aggy, (NS - 1) * sz0, szL)
        wout(accz, aggz, (NS - 1) * sz0, szL)


@functools.cache
def _scatter3():
    vN = jax.ShapeDtypeStruct((NC * N,), _f32)
    bC = pltpu.VMEM((NSLOT, CH), _f32)
    aN = pltpu.VMEM_SHARED((N,), _f32)
    return pl.kernel(
        _sc_scatter3_body,
        out_type=[vN, vN, vN],
        mesh=_mesh(),
        scratch_types=[
            pltpu.VMEM((NSLOT, CH), jnp.int32),
            (bC, bC, bC),
            aN, aN, aN,
            pltpu.VMEM((640,), _f32),
            ([pltpu.SemaphoreType.DMA] * NSLOT, [pltpu.SemaphoreType.DMA] * NSLOT),
        ],
    )


BN = 2000
BE = 3200
BEL = BE // H


def _rows(bs, w):
    return pl.BlockSpec((bs, w), lambda i: (i, 0))


def _full(shape):
    return pl.BlockSpec(shape, lambda i: tuple(0 for _ in shape))


def _silu(v):
    return v * jax.nn.sigmoid(v)


def _dot(a, b):
    return jnp.dot(a, b, preferred_element_type=_f32)


def _tc_proj_body(h_ref, wr_ref, wc_ref, b_ref, pr_ref, pc_ref):
    hb = h_ref[...]
    pr_ref[...] = _dot(hb, wr_ref[...]) + b_ref[...]
    pc_ref[...] = _dot(hb, wc_ref[...])


def _proj(h, wr, wc, b):
    return pl.pallas_call(
        _tc_proj_body,
        grid=(N // BN,),
        in_specs=[_rows(BN, H), _full((H, H)), _full((H, H)), _full((1, H))],
        out_specs=[_rows(BN, H), _rows(BN, H)],
        out_shape=[jax.ShapeDtypeStruct((N, H), _f32)] * 2,
    )(h, wr, wc, b)


def _tc_geom_body(xr_ref, yr_ref, zr_ref, xc_ref, yc_ref, zc_ref,
                  rad_ref, cnx_ref, cny_ref, cnz_ref):
    cdx = xr_ref[...] - xc_ref[...]
    cdy = yr_ref[...] - yc_ref[...]
    cdz = zr_ref[...] - zc_ref[...]
    radial = cdx * cdx + cdy * cdy + cdz * cdz
    inv = 1.0 / (jnp.sqrt(radial + 1e-8) + 1.0)
    rad_ref[...] = radial
    cnx_ref[...] = cdx * inv
    cny_ref[...] = cdy * inv
    cnz_ref[...] = cdz * inv


def _geom(xr, yr, zr, xc, yc, zc):
    s = _full((ER, H))
    o = jax.ShapeDtypeStruct((ER, H), _f32)
    return pl.pallas_call(
        _tc_geom_body,
        grid=(1,),
        in_specs=[s] * 6,
        out_specs=[s] * 4,
        out_shape=[o] * 4,
    )(xr, yr, zr, xc, yc, zc)


def _eterm(e2blk, wre):
    return lax.dot_general(e2blk, wre, (((0,), (0,)), ((), ())),
                           preferred_element_type=_f32)


def _tc_edge_gcl_body(gr_ref, gc_ref, e2_ref, w2_ref, b2_ref, wa_ref, ba_ref,
                      wre_ref, out_ref):
    v = gr_ref[...] + gc_ref[...] + _eterm(e2_ref[...], wre_ref[...])
    m1 = _silu(v)
    mm = _dot(m1, w2_ref[...]) + b2_ref[...]
    m = _silu(mm)
    att = jax.nn.sigmoid(_dot(m, wa_ref[...]) + ba_ref[...])
    out_ref[...] = m * att


def _edge_gcl(gr, gc, e2, w2, b2, wa, ba, wre, half):
    nb = EH // BE
    return pl.pallas_call(
        _tc_edge_gcl_body,
        grid=(nb,),
        in_specs=[_rows(BE, H), _rows(BE, H),
                  pl.BlockSpec((2, BE), lambda i: (0, i + half * nb)),
                  _full((H, H)), _full((1, H)), _full((H, 1)), _full((1, 1)),
                  _full((2, H))],
        out_specs=_rows(BE, H),
        out_shape=jax.ShapeDtypeStruct((EH, H), _f32),
    )(gr, gc, e2, w2, b2, wa, ba, wre)


def _tc_edge_equiv_body(gr_ref, gc_ref, e2_ref, w2_ref, b2_ref, w3_ref,
                        wre_ref, t_ref):
    v = gr_ref[...] + gc_ref[...] + _eterm(e2_ref[...], wre_ref[...])
    t1 = _silu(v)
    t2 = _silu(_dot(t1, w2_ref[...]) + b2_ref[...])
    t_ref[...] = lax.dot_general(w3_ref[...], t2, (((0,), (1,)), ((), ())),
                                 preferred_element_type=_f32)


def _edge_equiv(gr, gc, e2, w2, b2, w3, wre, half):
    nb = EH // BE
    return pl.pallas_call(
        _tc_edge_equiv_body,
        grid=(nb,),
        in_specs=[_rows(BE, H), _rows(BE, H),
                  pl.BlockSpec((2, BE), lambda i: (0, i + half * nb)),
                  _full((H, H)), _full((1, H)), _full((H, 1)), _full((2, H))],
        out_specs=pl.BlockSpec((1, BE), lambda i: (0, i)),
        out_shape=jax.ShapeDtypeStruct((1, EH), _f32),
    )(gr, gc, e2, w2, b2, w3, wre)


def _tc_trans_body(t_ref, cnx_ref, cny_ref, cnz_ref, tx_ref, ty_ref, tz_ref):
    t = t_ref[...]
    tx_ref[...] = cnx_ref[...] * t
    ty_ref[...] = cny_ref[...] * t
    tz_ref[...] = cnz_ref[...] * t


def _trans(t2d, cnx, cny, cnz):
    s = _full((ER, H))
    o = jax.ShapeDtypeStruct((ER, H), _f32)
    return pl.pallas_call(
        _tc_trans_body,
        grid=(1,),
        in_specs=[s] * 4,
        out_specs=[s] * 3,
        out_shape=[o] * 3,
    )(t2d, cnx, cny, cnz)


def _tc_node_body(h_ref, a0_ref, a1_ref, a2_ref, a3_ref, wna_ref, wnb_ref,
                  bn1_ref, wn2_ref, bn2_ref, wrn_ref, wcn_ref, brn_ref,
                  hout_ref, pr_ref, pc_ref):
    hb = h_ref[...]
    agg = (a0_ref[...] + a1_ref[...]) + (a2_ref[...] + a3_ref[...])
    pre = _dot(hb, wna_ref[...]) + _dot(agg, wnb_ref[...]) + bn1_ref[...]
    n1 = _silu(pre)
    ho = hb + _dot(n1, wn2_ref[...]) + bn2_ref[...]
    hout_ref[...] = ho
    pr_ref[...] = _dot(ho, wrn_ref[...]) + brn_ref[...]
    pc_ref[...] = _dot(ho, wcn_ref[...])


def _node(h, a0, a1, a2, a3, wna, wnb, bn1, wn2, bn2, wrn, wcn, brn):
    return pl.pallas_call(
        _tc_node_body,
        grid=(N // BN,),
        in_specs=[_rows(BN, H)] * 5 +
                 [_full((H, H)), _full((H, H)), _full((1, H)),
                  _full((H, H)), _full((1, H)),
                  _full((H, H)), _full((H, H)), _full((1, H))],
        out_specs=[_rows(BN, H)] * 3,
        out_shape=[jax.ShapeDtypeStruct((N, H), _f32)] * 3,
    )(h, a0, a1, a2, a3, wna, wnb, bn1, wn2, bn2, wrn, wcn, brn)


def _tc_final_body(x_ref, ax0, ax1, ay0, ay1, az0, az1, xout_ref):
    agg = jnp.concatenate([ax0[...] + ax1[...], ay0[...] + ay1[...],
                           az0[...] + az1[...]], axis=1)
    xout_ref[...] = x_ref[...] + agg * 0.01


def _final(x, ax0, ax1, ay0, ay1, az0, az1):
    return pl.pallas_call(
        _tc_final_body,
        grid=(N // BN,),
        in_specs=[_rows(BN, 3)] + [_rows(BN, 1)] * 6,
        out_specs=_rows(BN, 3),
        out_shape=jax.ShapeDtypeStruct((N, 3), _f32),
    )(x, ax0, ax1, ay0, ay1, az0, az1)


def _split_edge_w(lin):
    w = lin["w"]
    wre = jnp.stack([w[2 * H], w[2 * H + 1]], axis=0)
    return w[:H], w[H:2 * H], wre, lin["b"].reshape(1, H)


def kernel(h, x, edge_index, edge_attr, params):
    row = edge_index[0]
    col = edge_index[1]
    xx = x[:, 0]
    xy = x[:, 1]
    xz = x[:, 2]

    g0, g1, pe = params["gcl0"], params["gcl1"], params["equiv"]
    w1r0, w1c0, wre0, b10 = _split_edge_w(g0["edge1"])
    w1r1, w1c1, wre1, b11 = _split_edge_w(g1["edge1"])
    c1r, c1c, wree, bc1 = _split_edge_w(pe["c1"])

    zeros_h = jnp.zeros((N, H), _f32)

    def node_w(g):
        wn1 = g["node1"]["w"]
        return (wn1[:H], wn1[H:] * 0.01, g["node1"]["b"].reshape(1, H),
                g["node2"]["w"], g["node2"]["b"].reshape(1, H))

    pr0, pc0 = _proj(h, w1r0, w1c0, b10)
    oxr, oyr, ozr, oxc, oyc, ozc = _gather0()(xx, xy, xz, row, col)
    gr0a, gc0a = _gather2s(0)(pr0, pc0, row, col)
    gr0b, gc0b = _gather2s(EH)(pr0, pc0, row, col)
    rad2, cnx2, cny2, cnz2 = _geom(
        oxr.reshape(ER, H), oyr.reshape(ER, H), ozr.reshape(ER, H),
        oxc.reshape(ER, H), oyc.reshape(ER, H), ozc.reshape(ER, H))
    e2 = jnp.stack([rad2.reshape(E), edge_attr.reshape(E)], axis=0)
    w20 = g0["edge2"]["w"]
    b20 = g0["edge2"]["b"].reshape(1, H)
    wa0 = g0["att"]["w"]
    ba0 = g0["att"]["b"].reshape(1, 1)
    out0a = _edge_gcl(gr0a, gc0a, e2, w20, b20, wa0, ba0, wre0, 0)
    agg0a = _scatter_h(out0a, row, zeros_h, 0)
    out0b = _edge_gcl(gr0b, gc0b, e2, w20, b20, wa0, ba0, wre0, 1)
    agg0b = _scatter_h(out0b, row, zeros_h, EH)
    wna, wnb, bn1, wn2, bn2 = node_w(g0)
    h1, pr1, pc1 = _node(h, agg0a[0], agg0a[1], agg0b[0], agg0b[1],
                         wna, wnb, bn1, wn2, bn2, w1r1, w1c1, b11)

    gr1a, gc1a = _gather2s(0)(pr1, pc1, row, col)
    gr1b, gc1b = _gather2s(EH)(pr1, pc1, row, col)
    w21 = g1["edge2"]["w"]
    b21 = g1["edge2"]["b"].reshape(1, H)
    wa1 = g1["att"]["w"]
    ba1 = g1["att"]["b"].reshape(1, 1)
    out1a = _edge_gcl(gr1a, gc1a, e2, w21, b21, wa1, ba1, wre1, 0)
    agg1a = _scatter_h(out1a, row, zeros_h, 0)
    out1b = _edge_gcl(gr1b, gc1b, e2, w21, b21, wa1, ba1, wre1, 1)
    agg1b = _scatter_h(out1b, row, zeros_h, EH)
    wna, wnb, bn1, wn2, bn2 = node_w(g1)
    h2, qr, qc = _node(h1, agg1a[0], agg1a[1], agg1b[0], agg1b[1],
                       wna, wnb, bn1, wn2, bn2, c1r, c1c, bc1)

    gqra, gqca = _gather2s(0)(qr, qc, row, col)
    gqrb, gqcb = _gather2s(EH)(qr, qc, row, col)
    c2w = pe["c2"]["w"]
    c2b = pe["c2"]["b"].reshape(1, H)
    t1a = _edge_equiv(gqra, gqca, e2, c2w, c2b, pe["c3w"], wree, 0)
    t1b = _edge_equiv(gqrb, gqcb, e2, c2w, c2b, pe["c3w"], wree, 1)
    t1 = jnp.concatenate([t1a.reshape(EH), t1b.reshape(EH)])
    tx2, ty2, tz2 = _trans(t1.reshape(ER, H), cnx2, cny2, cnz2)
    zeros_1 = jnp.zeros((N,), _f32)
    ax, ay, az = _scatter3()(tx2.reshape(E), ty2.reshape(E), tz2.reshape(E),
                             row, zeros_1)
    x_out = _final(x,
                   ax[:N].reshape(N, 1), ax[N:].reshape(N, 1),
                   ay[:N].reshape(N, 1), ay[N:].reshape(N, 1),
                   az[:N].reshape(N, 1), az[N:].reshape(N, 1))

    return (h2, x_out)


# speedup vs baseline: 6.5587x; 1.0003x over previous
import functools

import jax
import jax.numpy as jnp
from jax import lax
from jax.experimental import pallas as pl
from jax.experimental.pallas import tpu as pltpu
import jax.experimental.pallas.tpu_sc as plsc

N = 10000
E = 320000
H = 128
ER = E // H

NC = 2
NS = 16
NW = NC * NS
EPW = E // NW
CH = 40
NCHUNK = EPW // CH
NSLOT = 5
NGRP = NCHUNK // NSLOT
EH = E // 2
EPW2 = EH // NW
NCHUNK2 = EPW2 // CH
NGRP2 = NCHUNK2 // NSLOT


def _mesh():
    return plsc.VectorSubcoreMesh(
        core_axis_name="c", subcore_axis_name="s",
        num_cores=NC, num_subcores=NS)


_f32 = jnp.float32


def _sc_gather0_body(xx, xy, xz, row, col,
                     oxr, oyr, ozr, oxc, oyc, ozc,
                     idx_r, idx_c, bufs, sems):
    wid = lax.axis_index("s") * NC + lax.axis_index("c")
    base = wid * EPW
    pltpu.sync_copy(row.at[pl.ds(base, EPW)], idx_r)
    pltpu.sync_copy(col.at[pl.ds(base, EPW)], idx_c)

    bxr, byr, bzr, bxc, byc, bzc = bufs
    gsem, ssem = sems

    def gops(i, s):
        ir = idx_r.at[pl.ds(i * CH, CH)]
        ic = idx_c.at[pl.ds(i * CH, CH)]
        return [
            (xx.at[ir], bxr.at[s]), (xy.at[ir], byr.at[s]),
            (xz.at[ir], bzr.at[s]),
            (xx.at[ic], bxc.at[s]), (xy.at[ic], byc.at[s]),
            (xz.at[ic], bzc.at[s]),
        ]

    def sops(s, off):
        d = pl.ds(off, CH)
        return [
            (bxr.at[s], oxr.at[d]), (byr.at[s], oyr.at[d]),
            (bzr.at[s], ozr.at[d]),
            (bxc.at[s], oxc.at[d]), (byc.at[s], oyc.at[d]),
            (bzc.at[s], ozc.at[d]),
        ]

    def fire(i, s):
        for a, b in gops(i, s):
            pltpu.async_copy(a, b, gsem[s])

    for s in range(NSLOT):
        fire(s, s)

    def group(g, carry):
        for s in range(NSLOT):
            i = g * NSLOT + s
            off = base + i * CH
            for a, b in gops(i, s):
                pltpu.make_async_copy(a, b, gsem[s]).wait()
            for a, b in sops(s, off):
                pltpu.async_copy(a, b, ssem[s])
            for a, b in sops(s, off):
                pltpu.make_async_copy(a, b, ssem[s]).wait()

            @pl.when(g < NGRP - 1)
            def _():
                fire(i + NSLOT, s)
        return carry

    lax.fori_loop(0, NGRP, group, 0)


@functools.cache
def _gather0():
    vE = jax.ShapeDtypeStruct((E,), _f32)
    bC = pltpu.VMEM((NSLOT, CH), _f32)
    return pl.kernel(
        _sc_gather0_body,
        out_type=[vE, vE, vE, vE, vE, vE],
        mesh=_mesh(),
        scratch_types=[
            pltpu.VMEM((EPW,), jnp.int32),
            pltpu.VMEM((EPW,), jnp.int32),
            (bC, bC, bC, bC, bC, bC),
            ([pltpu.SemaphoreType.DMA] * NSLOT, [pltpu.SemaphoreType.DMA] * NSLOT),
        ],
    )


def _make_gather2s_body(eoff):
    def body(pr, pc, row, col, gr, gc, idx, buf, acc, sems):
        cid = lax.axis_index("c")
        sid = lax.axis_index("s")
        gsem, ssem = sems

        @pl.when((sid == 0) & (cid == 0))
        def _():
            pltpu.sync_copy(pr, acc)

        @pl.when((sid == 0) & (cid == 1))
        def _():
            pltpu.sync_copy(pc, acc)

        plsc.subcore_barrier()

        def pipeline(idxarr, out):
            base = eoff + sid * (EH // NS)
            lbase = sid * (EH // NS)
            pltpu.sync_copy(idxarr.at[pl.ds(base, EPW)], idx)

            def fire(i, s):
                pltpu.async_copy(acc.at[idx.at[pl.ds(i * CH, CH)]], buf.at[s],
                                 gsem[s])

            for s in range(NSLOT):
                fire(s, s)

            def group(g, carry):
                for s in range(NSLOT):
                    i = g * NSLOT + s
                    off = lbase + i * CH
                    ii = idx.at[pl.ds(i * CH, CH)]
                    pltpu.make_async_copy(acc.at[ii], buf.at[s], gsem[s]).wait()
                    pltpu.async_copy(buf.at[s], out.at[pl.ds(off, CH)], ssem[s])
                    pltpu.make_async_copy(buf.at[s], out.at[pl.ds(off, CH)],
                                          ssem[s]).wait()

                    @pl.when(g < NGRP - 1)
                    def _():
                        fire(i + NSLOT, s)
                return carry

            lax.fori_loop(0, NGRP, group, 0)

        @pl.when(cid == 0)
        def _():
            pipeline(row, gr)

        @pl.when(cid == 1)
        def _():
            pipeline(col, gc)

    return body


@functools.cache
def _gather2s(eoff):
    return pl.kernel(
        _make_gather2s_body(eoff),
        out_type=[
            jax.ShapeDtypeStruct((EH, H), _f32),
            jax.ShapeDtypeStruct((EH, H), _f32),
        ],
        mesh=_mesh(),
        scratch_types=[
            pltpu.VMEM((EPW,), jnp.int32),
            pltpu.VMEM((NSLOT, CH, H), _f32),
            pltpu.VMEM_SHARED((N, H), _f32),
            ([pltpu.SemaphoreType.DMA] * NSLOT, [pltpu.SemaphoreType.DMA] * NSLOT),
        ],
    )


@functools.cache
def _make_scatter(eoff):
    W = H

    def body(vals, row, zeros, agg, idx, buf, acc, sems):
        cid = lax.axis_index("c")
        sid = lax.axis_index("s")
        wid = cid * NS + sid
        base = eoff + wid * EPW2
        lbase = wid * EPW2
        gsem, ssem = sems

        @pl.when(sid == 0)
        def _():
            pltpu.sync_copy(zeros, acc)

        plsc.subcore_barrier()

        def fire(i, s):
            pltpu.async_copy(row.at[pl.ds(base + i * CH, CH)], idx.at[s], gsem[s])
            pltpu.async_copy(vals.at[pl.ds(lbase + i * CH, CH)], buf.at[s],
                             gsem[s])

        for s in range(NSLOT):
            fire(s, s)

        def group(g, carry):
            for s in range(NSLOT):
                i = g * NSLOT + s
                pltpu.make_async_copy(
                    row.at[pl.ds(base + i * CH, CH)], idx.at[s], gsem[s]).wait()
                pltpu.make_async_copy(
                    vals.at[pl.ds(lbase + i * CH, CH)], buf.at[s], gsem[s]).wait()
                cp = pltpu.async_copy(buf.at[s], acc.at[idx.at[s]], ssem[s],
                                      add=True)
                cp.wait()

                @pl.when(g < NGRP2 - 1)
                def _():
                    fire(i + NSLOT, s)
            return carry

        lax.fori_loop(0, NGRP2, group, 0)
        plsc.subcore_barrier()

        @pl.when(sid < NS - 1)
        def _():
            pltpu.sync_copy(acc.at[pl.ds(sid * 624, 624)],
                            agg.at[cid, pl.ds(sid * 624, 624)])

        @pl.when(sid == NS - 1)
        def _():
            pltpu.sync_copy(acc.at[pl.ds((NS - 1) * 624, N - (NS - 1) * 624)],
                            agg.at[cid, pl.ds((NS - 1) * 624, N - (NS - 1) * 624)])

    return pl.kernel(
        body,
        out_type=jax.ShapeDtypeStruct((NC, N, W), _f32),
        mesh=_mesh(),
        scratch_types=[
            pltpu.VMEM((NSLOT, CH), jnp.int32),
            pltpu.VMEM((NSLOT, CH, W), _f32),
            pltpu.VMEM_SHARED((N, W), _f32),
            ([pltpu.SemaphoreType.DMA] * NSLOT, [pltpu.SemaphoreType.DMA] * NSLOT),
        ],
    )


def _scatter_h(vals, row, zeros, eoff):
    return _make_scatter(eoff)(vals, row, zeros)


def _sc_scatter3_body(tx, ty, tz, row, zeros1, aggx, aggy, aggz,
                      idx, bufs, accx, accy, accz, bo, sems):
    cid = lax.axis_index("c")
    sid = lax.axis_index("s")
    wid = cid * NS + sid
    base = wid * EPW
    bx, by, bz = bufs
    gsem, ssem = sems

    @pl.when(sid == 0)
    def _():
        pltpu.sync_copy(zeros1, accx)
        pltpu.sync_copy(zeros1, accy)
        pltpu.sync_copy(zeros1, accz)

    plsc.subcore_barrier()

    def fire(i, s):
        off = base + i * CH
        pltpu.async_copy(row.at[pl.ds(off, CH)], idx.at[s], gsem[s])
        pltpu.async_copy(tx.at[pl.ds(off, CH)], bx.at[s], gsem[s])
        pltpu.async_copy(ty.at[pl.ds(off, CH)], by.at[s], gsem[s])
        pltpu.async_copy(tz.at[pl.ds(off, CH)], bz.at[s], gsem[s])

    for s in range(NSLOT):
        fire(s, s)

    def group(g, carry):
        for s in range(NSLOT):
            i = g * NSLOT + s
            off = base + i * CH
            pltpu.make_async_copy(row.at[pl.ds(off, CH)], idx.at[s], gsem[s]).wait()
            pltpu.make_async_copy(tx.at[pl.ds(off, CH)], bx.at[s], gsem[s]).wait()
            pltpu.make_async_copy(ty.at[pl.ds(off, CH)], by.at[s], gsem[s]).wait()
            pltpu.make_async_copy(tz.at[pl.ds(off, CH)], bz.at[s], gsem[s]).wait()
            pltpu.async_copy(bx.at[s], accx.at[idx.at[s]], ssem[s], add=True)
            pltpu.async_copy(by.at[s], accy.at[idx.at[s]], ssem[s], add=True)
            pltpu.async_copy(bz.at[s], accz.at[idx.at[s]], ssem[s], add=True)
            pltpu.make_async_copy(bx.at[s], accx.at[idx.at[s]], ssem[s]).wait()
            pltpu.make_async_copy(by.at[s], accy.at[idx.at[s]], ssem[s]).wait()
            pltpu.make_async_copy(bz.at[s], accz.at[idx.at[s]], ssem[s]).wait()

            @pl.when(g < NGRP - 1)
            def _():
                fire(i + NSLOT, s)
        return carry

    lax.fori_loop(0, NGRP, group, 0)
    plsc.subcore_barrier()

    sz0 = 624
    szL = N - (NS - 1) * sz0

    def wout(acc, agg, start, sz):
        pltpu.sync_copy(acc.at[pl.ds(start, sz)], bo.at[pl.ds(0, sz)])
        pltpu.sync_copy(bo.at[pl.ds(0, sz)], agg.at[pl.ds(cid * N + start, sz)])

    @pl.when(sid < NS - 1)
    def _():
        wout(accx, aggx, sid * sz0, sz0)
        wout(accy, aggy, sid * sz0, sz0)
        wout(accz, aggz, sid * sz0, sz0)

    @pl.when(sid == NS - 1)
    def _():
        wout(accx, aggx, (NS - 1) * sz0, szL)
        wout(accy, aggy, (NS - 1) * sz0, szL)
        wout(accz, aggz, (NS - 1) * sz0, szL)


@functools.cache
def _scatter3():
    vN = jax.ShapeDtypeStruct((NC * N,), _f32)
    bC = pltpu.VMEM((NSLOT, CH), _f32)
    aN = pltpu.VMEM_SHARED((N,), _f32)
    return pl.kernel(
        _sc_scatter3_body,
        out_type=[vN, vN, vN],
        mesh=_mesh(),
        scratch_types=[
            pltpu.VMEM((NSLOT, CH), jnp.int32),
            (bC, bC, bC),
            aN, aN, aN,
            pltpu.VMEM((640,), _f32),
            ([pltpu.SemaphoreType.DMA] * NSLOT, [pltpu.SemaphoreType.DMA] * NSLOT),
        ],
    )


BN = 2000
BE = 3200
BEL = BE // H


def _rows(bs, w):
    return pl.BlockSpec((bs, w), lambda i: (i, 0))


def _full(shape):
    return pl.BlockSpec(shape, lambda i: tuple(0 for _ in shape))


def _silu(v):
    return v * jax.nn.sigmoid(v)


def _dot(a, b):
    return jnp.dot(a, b, preferred_element_type=_f32)


def _tc_proj_body(h_ref, wr_ref, wc_ref, b_ref, pr_ref, pc_ref):
    hb = h_ref[...]
    pr_ref[...] = _dot(hb, wr_ref[...]) + b_ref[...]
    pc_ref[...] = _dot(hb, wc_ref[...])


def _proj(h, wr, wc, b):
    return pl.pallas_call(
        _tc_proj_body,
        grid=(N // BN,),
        in_specs=[_rows(BN, H), _full((H, H)), _full((H, H)), _full((1, H))],
        out_specs=[_rows(BN, H), _rows(BN, H)],
        out_shape=[jax.ShapeDtypeStruct((N, H), _f32)] * 2,
    )(h, wr, wc, b)


def _tc_geom_body(xr_ref, yr_ref, zr_ref, xc_ref, yc_ref, zc_ref,
                  rad_ref, cnx_ref, cny_ref, cnz_ref):
    cdx = xr_ref[...] - xc_ref[...]
    cdy = yr_ref[...] - yc_ref[...]
    cdz = zr_ref[...] - zc_ref[...]
    radial = cdx * cdx + cdy * cdy + cdz * cdz
    inv = 1.0 / (jnp.sqrt(radial + 1e-8) + 1.0)
    rad_ref[...] = radial
    cnx_ref[...] = cdx * inv
    cny_ref[...] = cdy * inv
    cnz_ref[...] = cdz * inv


def _geom(xr, yr, zr, xc, yc, zc):
    s = _full((ER, H))
    o = jax.ShapeDtypeStruct((ER, H), _f32)
    return pl.pallas_call(
        _tc_geom_body,
        grid=(1,),
        in_specs=[s] * 6,
        out_specs=[s] * 4,
        out_shape=[o] * 4,
    )(xr, yr, zr, xc, yc, zc)


def _eterm(e2blk, wre):
    return lax.dot_general(e2blk, wre, (((0,), (0,)), ((), ())),
                           preferred_element_type=_f32)


def _tc_edge_gcl_body(gr_ref, gc_ref, e2_ref, w2_ref, b2_ref, wa_ref, ba_ref,
                      wre_ref, out_ref):
    v = gr_ref[...] + gc_ref[...] + _eterm(e2_ref[...], wre_ref[...])
    m1 = _silu(v)
    mm = _dot(m1, w2_ref[...]) + b2_ref[...]
    m = _silu(mm)
    att = jax.nn.sigmoid(_dot(m, wa_ref[...]) + ba_ref[...])
    out_ref[...] = m * att


def _edge_gcl(gr, gc, e2, w2, b2, wa, ba, wre, half):
    nb = EH // BE
    return pl.pallas_call(
        _tc_edge_gcl_body,
        grid=(nb,),
        in_specs=[_rows(BE, H), _rows(BE, H),
                  pl.BlockSpec((2, BE), lambda i: (0, i + half * nb)),
                  _full((H, H)), _full((1, H)), _full((H, 1)), _full((1, 1)),
                  _full((2, H))],
        out_specs=_rows(BE, H),
        out_shape=jax.ShapeDtypeStruct((EH, H), _f32),
    )(gr, gc, e2, w2, b2, wa, ba, wre)


def _tc_edge_equiv_body(gr_ref, gc_ref, e2_ref, w2_ref, b2_ref, w3_ref,
                        wre_ref, t_ref):
    v = gr_ref[...] + gc_ref[...] + _eterm(e2_ref[...], wre_ref[...])
    t1 = _silu(v)
    t2 = _silu(_dot(t1, w2_ref[...]) + b2_ref[...])
    t_ref[...] = lax.dot_general(w3_ref[...], t2, (((0,), (1,)), ((), ())),
                                 preferred_element_type=_f32)


def _edge_equiv(gr, gc, e2, w2, b2, w3, wre, half):
    nb = EH // BE
    return pl.pallas_call(
        _tc_edge_equiv_body,
        grid=(nb,),
        in_specs=[_rows(BE, H), _rows(BE, H),
                  pl.BlockSpec((2, BE), lambda i: (0, i + half * nb)),
                  _full((H, H)), _full((1, H)), _full((H, 1)), _full((2, H))],
        out_specs=pl.BlockSpec((1, BE), lambda i: (0, i)),
        out_shape=jax.ShapeDtypeStruct((1, EH), _f32),
    )(gr, gc, e2, w2, b2, w3, wre)


def _tc_trans_body(t_ref, cnx_ref, cny_ref, cnz_ref, tx_ref, ty_ref, tz_ref):
    t = t_ref[...]
    tx_ref[...] = cnx_ref[...] * t
    ty_ref[...] = cny_ref[...] * t
    tz_ref[...] = cnz_ref[...] * t


def _trans(t2d, cnx, cny, cnz):
    s = _full((ER, H))
    o = jax.ShapeDtypeStruct((ER, H), _f32)
    return pl.pallas_call(
        _tc_trans_body,
        grid=(1,),
        in_specs=[s] * 4,
        out_specs=[s] * 3,
        out_shape=[o] * 3,
    )(t2d, cnx, cny, cnz)


def _tc_node_body(h_ref, a0_ref, a1_ref, a2_ref, a3_ref, wna_ref, wnb_ref,
                  bn1_ref, wn2_ref, bn2_ref, wrn_ref, wcn_ref, brn_ref,
                  hout_ref, pr_ref, pc_ref):
    hb = h_ref[...]
    agg = (a0_ref[...] + a1_ref[...]) + (a2_ref[...] + a3_ref[...])
    pre = _dot(hb, wna_ref[...]) + _dot(agg, wnb_ref[...]) + bn1_ref[...]
    n1 = _silu(pre)
    ho = hb + _dot(n1, wn2_ref[...]) + bn2_ref[...]
    hout_ref[...] = ho
    pr_ref[...] = _dot(ho, wrn_ref[...]) + brn_ref[...]
    pc_ref[...] = _dot(ho, wcn_ref[...])


def _node(h, a0, a1, a2, a3, wna, wnb, bn1, wn2, bn2, wrn, wcn, brn):
    return pl.pallas_call(
        _tc_node_body,
        grid=(N // BN,),
        in_specs=[_rows(BN, H)] * 5 +
                 [_full((H, H)), _full((H, H)), _full((1, H)),
                  _full((H, H)), _full((1, H)),
                  _full((H, H)), _full((H, H)), _full((1, H))],
        out_specs=[_rows(BN, H)] * 3,
        out_shape=[jax.ShapeDtypeStruct((N, H), _f32)] * 3,
    )(h, a0, a1, a2, a3, wna, wnb, bn1, wn2, bn2, wrn, wcn, brn)


def _tc_final_body(x_ref, ax0, ax1, ay0, ay1, az0, az1, xout_ref):
    agg = jnp.concatenate([ax0[...] + ax1[...], ay0[...] + ay1[...],
                           az0[...] + az1[...]], axis=1)
    xout_ref[...] = x_ref[...] + agg * 0.01


def _final(x, ax0, ax1, ay0, ay1, az0, az1):
    return pl.pallas_call(
        _tc_final_body,
        grid=(N // BN,),
        in_specs=[_rows(BN, 3)] + [_rows(BN, 1)] * 6,
        out_specs=_rows(BN, 3),
        out_shape=jax.ShapeDtypeStruct((N, 3), _f32),
    )(x, ax0, ax1, ay0, ay1, az0, az1)


def _split_edge_w(lin):
    w = lin["w"]
    wre = jnp.stack([w[2 * H], w[2 * H + 1]], axis=0)
    return w[:H], w[H:2 * H], wre, lin["b"].reshape(1, H)


def kernel(h, x, edge_index, edge_attr, params):
    row = edge_index[0]
    col = edge_index[1]
    xx = x[:, 0]
    xy = x[:, 1]
    xz = x[:, 2]

    g0, g1, pe = params["gcl0"], params["gcl1"], params["equiv"]
    w1r0, w1c0, wre0, b10 = _split_edge_w(g0["edge1"])
    w1r1, w1c1, wre1, b11 = _split_edge_w(g1["edge1"])
    c1r, c1c, wree, bc1 = _split_edge_w(pe["c1"])

    zeros_h = jnp.zeros((N, H), _f32)

    def node_w(g):
        wn1 = g["node1"]["w"]
        return (wn1[:H], wn1[H:] * 0.01, g["node1"]["b"].reshape(1, H),
                g["node2"]["w"], g["node2"]["b"].reshape(1, H))

    pr0, pc0 = _proj(h, w1r0, w1c0, b10)
    oxr, oyr, ozr, oxc, oyc, ozc = _gather0()(xx, xy, xz, row, col)
    gr0a, gc0a = _gather2s(0)(pr0, pc0, row, col)
    gr0b, gc0b = _gather2s(EH)(pr0, pc0, row, col)
    rad2, cnx2, cny2, cnz2 = _geom(
        oxr.reshape(ER, H), oyr.reshape(ER, H), ozr.reshape(ER, H),
        oxc.reshape(ER, H), oyc.reshape(ER, H), ozc.reshape(ER, H))
    e2 = jnp.stack([rad2.reshape(E), edge_attr.reshape(E)], axis=0)
    w20 = g0["edge2"]["w"]
    b20 = g0["edge2"]["b"].reshape(1, H)
    wa0 = g0["att"]["w"]
    ba0 = g0["att"]["b"].reshape(1, 1)
    out0a = _edge_gcl(gr0a, gc0a, e2, w20, b20, wa0, ba0, wre0, 0)
    agg0a = _scatter_h(out0a, row, zeros_h, 0)
    out0b = _edge_gcl(gr0b, gc0b, e2, w20, b20, wa0, ba0, wre0, 1)
    agg0b = _scatter_h(out0b, row, zeros_h, EH)
    wna, wnb, bn1, wn2, bn2 = node_w(g0)
    h1, pr1, pc1 = _node(h, agg0a[0], agg0a[1], agg0b[0], agg0b[1],
                         wna, wnb, bn1, wn2, bn2, w1r1, w1c1, b11)

    gr1a, gc1a = _gather2s(0)(pr1, pc1, row, col)
    gr1b, gc1b = _gather2s(EH)(pr1, pc1, row, col)
    w21 = g1["edge2"]["w"]
    b21 = g1["edge2"]["b"].reshape(1, H)
    wa1 = g1["att"]["w"]
    ba1 = g1["att"]["b"].reshape(1, 1)
    out1a = _edge_gcl(gr1a, gc1a, e2, w21, b21, wa1, ba1, wre1, 0)
    agg1a = _scatter_h(out1a, row, zeros_h, 0)
    out1b = _edge_gcl(gr1b, gc1b, e2, w21, b21, wa1, ba1, wre1, 1)
    agg1b = _scatter_h(out1b, row, zeros_h, EH)
    wna, wnb, bn1, wn2, bn2 = node_w(g1)
    h2, qr, qc = _node(h1, agg1a[0], agg1a[1], agg1b[0], agg1b[1],
                       wna, wnb, bn1, wn2, bn2, c1r, c1c, bc1)

    gqra, gqca = _gather2s(0)(qr, qc, row, col)
    gqrb, gqcb = _gather2s(EH)(qr, qc, row, col)
    c2w = pe["c2"]["w"]
    c2b = pe["c2"]["b"].reshape(1, H)
    t1a = _edge_equiv(gqra, gqca, e2, c2w, c2b, pe["c3w"], wree, 0)
    t1b = _edge_equiv(gqrb, gqcb, e2, c2w, c2b, pe["c3w"], wree, 1)
    t1 = jnp.concatenate([t1a.reshape(EH), t1b.reshape(EH)])
    tx2, ty2, tz2 = _trans(t1.reshape(ER, H), cnx2, cny2, cnz2)
    zeros_1 = jnp.zeros((N,), _f32)
    ax, ay, az = _scatter3()(tx2.reshape(E), ty2.reshape(E), tz2.reshape(E),
                             row, zeros_1)
    x_out = _final(x,
                   ax[:N].reshape(N, 1), ax[N:].reshape(N, 1),
                   ay[:N].reshape(N, 1), ay[N:].reshape(N, 1),
                   az[:N].reshape(N, 1), az[N:].reshape(N, 1))

    return (h2, x_out)
